# Initial kernel scaffold; baseline (speedup 1.0000x reference)
import jax
import jax.numpy as jnp
from jax.experimental import pallas as pl


def kernel(x, edge_index, edge_attr, W1, b1, W2, b2, mW1, mb1, mW2, mb2, fcW, fcb):
    raise NotImplementedError("write your pallas kernel here")


# speedup vs baseline: 22.8599x; 22.8599x over previous
import functools

import jax
import jax.numpy as jnp
from jax import lax
from jax.experimental import pallas as pl
from jax.experimental.pallas import tpu as pltpu
from jax.experimental.pallas import tpu_sc as plsc

N = 10000
E = 320000
DF = 128
H = 16

NC, NS, L = 2, 16, 16
NW = NC * NS
NPAD = 10240
NPT = NPAD // NS
EPC = E
EPT = EPC // NS
CH = 2000
S = 125
KR = 8
RPT = EPT // S
NGRP = RPT // KR
EOUT_PT = E // NW
NOCH = EOUT_PT // CH


def _mesh():
    return plsc.VectorSubcoreMesh(core_axis_name="c", subcore_axis_name="s")


_SC_PARAMS = pltpu.CompilerParams(needs_layout_passes=False,
                                  use_tc_tiling_on_sc=False)


def _tc_lin1(x, W1):
    def body(x_ref, w_ref, o_ref):
        o_ref[...] = jnp.dot(x_ref[...], w_ref[...],
                             preferred_element_type=jnp.float32)
    return pl.pallas_call(
        body,
        out_shape=jax.ShapeDtypeStruct((N, H), jnp.float32),
    )(x, W1)


def _tc_edge_mlp(ea, mW1, mb1, mW2, f3, mb2, fcb):
    EB = 8000

    def body(ea_ref, w1_ref, b1_ref, w2_ref, f3_ref, b2_ref, fcb_ref, o_ref):
        t = jnp.dot(ea_ref[...], w1_ref[...],
                    preferred_element_type=jnp.float32) + b1_ref[...]
        t = jnp.maximum(t, 0.0)
        w3 = jnp.dot(w2_ref[...], f3_ref[...],
                     preferred_element_type=jnp.float32)
        cst = jnp.dot(b2_ref[...], f3_ref[...],
                      preferred_element_type=jnp.float32) + fcb_ref[...]
        o_ref[...] = jnp.dot(t, w3, preferred_element_type=jnp.float32) + cst

    return pl.pallas_call(
        body,
        grid=(E // EB,),
        in_specs=[
            pl.BlockSpec((EB, H), lambda i: (i, 0)),
            pl.BlockSpec((H, H), lambda i: (0, 0)),
            pl.BlockSpec((1, H), lambda i: (0, 0)),
            pl.BlockSpec((H, H), lambda i: (0, 0)),
            pl.BlockSpec((H, 1), lambda i: (0, 0)),
            pl.BlockSpec((1, H), lambda i: (0, 0)),
            pl.BlockSpec((1, 1), lambda i: (0, 0)),
        ],
        out_specs=pl.BlockSpec((EB, 1), lambda i: (i, 0)),
        out_shape=jax.ShapeDtypeStruct((E, 1), jnp.float32),
    )(ea, mW1, mb1, mW2, f3, mb2, fcb)


@functools.partial(
    pl.kernel,
    out_type=jax.ShapeDtypeStruct((E,), jnp.float32),
    mesh=_mesh(),
    compiler_params=_SC_PARAMS,
    scratch_types=[
        pltpu.VMEM((CH,), jnp.int32),
        pltpu.VMEM((CH,), jnp.int32),
        pltpu.VMEM((CH,), jnp.float32),
        pltpu.VMEM((CH,), jnp.float32),
        pltpu.VMEM((KR, S), jnp.int32),
        pltpu.VMEM((KR, S), jnp.int32),
        pltpu.VMEM((KR, S, H), jnp.float32),
        pltpu.VMEM((NPT, H), jnp.float32),
        pltpu.VMEM((NPT, H), jnp.float32),
        pltpu.VMEM((NPT, H), jnp.float32),
        pltpu.VMEM((NPT,), jnp.float32),
        pltpu.VMEM((NPT,), jnp.float32),
        pltpu.VMEM((NPT,), jnp.float32),
        pltpu.VMEM((NPT,), jnp.float32),
        pltpu.VMEM((H * H,), jnp.float32),
        pltpu.VMEM((H,), jnp.float32),
        pltpu.VMEM((H,), jnp.float32),
        pltpu.VMEM((H,), jnp.float32),
        pltpu.VMEM((H,), jnp.float32),
        pltpu.VMEM((NPAD,), jnp.float32),
        pltpu.VMEM((NPAD,), jnp.float32),
        pltpu.VMEM_SHARED((NS * NPAD,), jnp.float32),
        pltpu.VMEM_SHARED((NPAD, H), jnp.float32),
        pltpu.VMEM_SHARED((NPAD, H), jnp.float32),
        pltpu.VMEM_SHARED((NPAD, H), jnp.float32),
        pltpu.VMEM_SHARED((NPAD,), jnp.float32),
        pltpu.VMEM_SHARED((NPAD,), jnp.float32),
        pltpu.SemaphoreType.DMA,
        pltpu.SemaphoreType.DMA,
    ],
)
def _sc_mega(srcT_hbm, dstT_hbm, src_hbm, dst_hbm, lin1_hbm, zeros_hbm,
             w2_hbm, b1_hbm, b2_hbm, f1_hbm, f2_hbm, c_hbm, pred_hbm,
             idx_v, didx1_v, c_v, p_v, sidx_v, didx_v, rows_v,
             nbuf_v, g1s_v, g2s_v, dinv_v, tmp_v, av_v, bv_v,
             w2_v, b1_v, b2_v, f1_v, f2_v, a_full, b_full,
             part_sh, g1_sh, acc1_sh, g2_sh, a_sh, b_sh,
             gsem, ssem):
    cc = lax.axis_index("c")
    ss = lax.axis_index("s")
    wid = ss * NC + cc
    nbase = ss * NPT
    zeros = jnp.zeros((L,), jnp.float32)
    ones = jnp.ones((L,), jnp.float32)

    pltpu.sync_copy(w2_hbm, w2_v)
    pltpu.sync_copy(b1_hbm, b1_v)
    pltpu.sync_copy(b2_hbm, b2_v)
    pltpu.sync_copy(f1_hbm, f1_v)
    pltpu.sync_copy(f2_hbm, f2_v)
    pltpu.sync_copy(zeros_hbm.at[pl.ds(nbase, NPT)],
                    acc1_sh.at[pl.ds(nbase, NPT)])

    def zb(i, c):
        a_full[pl.ds(i * L, L)] = zeros
        return c
    lax.fori_loop(0, NPAD // L, zb, 0)

    def dchunk(ci, c):
        pltpu.sync_copy(dst_hbm.at[pl.ds(ss * EPT + ci * CH, CH)], idx_v)

        def scat(j, c2):
            plsc.addupdate_scatter(a_full, [idx_v[pl.ds(j * L, L)]], ones)
            return c2
        lax.fori_loop(0, CH // L, scat, 0)
        return c
    lax.fori_loop(0, EPT // CH, dchunk, 0)
    pltpu.sync_copy(a_full, part_sh.at[pl.ds(ss * NPAD, NPAD)])
    plsc.subcore_barrier()

    def z2(i, c):
        dinv_v[pl.ds(i * L, L)] = zeros
        return c
    lax.fori_loop(0, NPT // L, z2, 0)
    for t in range(NS):
        pltpu.sync_copy(part_sh.at[pl.ds(t * NPAD + nbase, NPT)], tmp_v)

        def ab(i, c):
            sl = pl.ds(i * L, L)
            dinv_v[sl] = dinv_v[sl] + tmp_v[sl]
            return c
        lax.fori_loop(0, NPT // L, ab, 0)

    def newton(i, c):
        sl = pl.ds(i * L, L)
        xv = dinv_v[sl] + 1.0
        iv = plsc.bitcast(xv, jnp.int32)
        iv = 0x5F3759DF - lax.shift_right_logical(iv, 1)
        y = plsc.bitcast(iv, jnp.float32)
        y = y * (1.5 - 0.5 * xv * y * y)
        y = y * (1.5 - 0.5 * xv * y * y)
        y = y * (1.5 - 0.5 * xv * y * y)
        dinv_v[sl] = y
        return c
    lax.fori_loop(0, NPT // L, newton, 0)

    pltpu.sync_copy(lin1_hbm.at[pl.ds(nbase, NPT)], g1s_v)

    def g1b(g, c):
        dvec = dinv_v[pl.ds(g * L, L)]
        for t in range(L):
            n = g * L + t
            g1s_v[n] = g1s_v[n] * dvec[t]
        return c
    lax.fori_loop(0, NPT // L, g1b, 0)
    pltpu.sync_copy(g1s_v, g1_sh.at[pl.ds(nbase, NPT)])
    plsc.subcore_barrier()

    def aggregate(g_sh, acc_sh):
        def grp(ci, cr):
            rb = ss * RPT + ci * KR
            pltpu.sync_copy(srcT_hbm.at[pl.ds(rb, KR)], sidx_v)
            pltpu.sync_copy(dstT_hbm.at[pl.ds(rb, KR)], didx_v)
            hs = []
            for j in range(KR):
                hs.append(pltpu.async_copy(
                    g_sh.at[sidx_v.at[j]], rows_v.at[j], gsem))
            sc = []
            for j in range(KR):
                hs[j].wait()
                sc.append(pltpu.async_copy(
                    rows_v.at[j], acc_sh.at[didx_v.at[j]], ssem, add=True))
            for j in range(KR):
                sc[j].wait()
            return cr
        lax.fori_loop(0, NGRP, grp, 0)

    aggregate(g1_sh, acc1_sh)
    plsc.subcore_barrier()

    pltpu.sync_copy(acc1_sh.at[pl.ds(nbase, NPT)], nbuf_v)
    pltpu.sync_copy(zeros_hbm.at[pl.ds(nbase, NPT)],
                    acc1_sh.at[pl.ds(nbase, NPT)])
    b1vec = b1_v[...]
    w2rows = [w2_v[pl.ds(k * H, H)] for k in range(H)]

    def s4b(g, c):
        dvec = dinv_v[pl.ds(g * L, L)]
        for t in range(L):
            n = g * L + t
            s = dvec[t]
            acc = nbuf_v[n] + g1s_v[n]
            h1 = jnp.maximum(s * acc + b1vec, 0.0)
            lin2 = h1[0] * w2rows[0]
            for k in range(1, H):
                lin2 = lin2 + h1[k] * w2rows[k]
            g2s_v[n] = s * lin2
        return c
    lax.fori_loop(0, NPT // L, s4b, 0)
    pltpu.sync_copy(g2s_v, g2_sh.at[pl.ds(nbase, NPT)])
    plsc.subcore_barrier()

    aggregate(g2_sh, acc1_sh)
    plsc.subcore_barrier()

    pltpu.sync_copy(acc1_sh.at[pl.ds(nbase, NPT)], nbuf_v)
    b2vec = b2_v[...]
    f1vec = f1_v[...]
    f2vec = f2_v[...]
    lanes = lax.iota(jnp.int32, L)

    def s6b(g, c):
        dvec = dinv_v[pl.ds(g * L, L)]
        a_acc = jnp.zeros((L,), jnp.float32)
        b_acc = jnp.zeros((L,), jnp.float32)
        for t in range(L):
            n = g * L + t
            h2 = dvec[t] * (nbuf_v[n] + g2s_v[n]) + b2vec
            a_acc = jnp.where(lanes == t, jnp.sum(h2 * f1vec), a_acc)
            b_acc = jnp.where(lanes == t, jnp.sum(h2 * f2vec), b_acc)
        sl = pl.ds(g * L, L)
        av_v[sl] = a_acc
        bv_v[sl] = b_acc
        return c
    lax.fori_loop(0, NPT // L, s6b, 0)
    pltpu.sync_copy(av_v, a_sh.at[pl.ds(nbase, NPT)])
    pltpu.sync_copy(bv_v, b_sh.at[pl.ds(nbase, NPT)])
    plsc.subcore_barrier()

    pltpu.sync_copy(a_sh, a_full)
    pltpu.sync_copy(b_sh, b_full)
    ebase = wid * EOUT_PT

    def ochunk(ci, cr):
        base = ebase + ci * CH
        pltpu.sync_copy(src_hbm.at[pl.ds(base, CH)], idx_v)
        pltpu.sync_copy(dst_hbm.at[pl.ds(base, CH)], didx1_v)
        pltpu.sync_copy(c_hbm.at[pl.ds(base, CH)], c_v)

        def jb(j, c2):
            sl = pl.ds(j * L, L)
            av = plsc.load_gather(a_full, [idx_v[sl]])
            bv = plsc.load_gather(b_full, [didx1_v[sl]])
            p_v[sl] = av + bv + c_v[sl]
            return c2
        lax.fori_loop(0, CH // L, jb, 0)
        pltpu.sync_copy(p_v, pred_hbm.at[pl.ds(base, CH)])
        return cr
    lax.fori_loop(0, NOCH, ochunk, 0)


def kernel(x, edge_index, edge_attr, W1, b1, W2, b2, mW1, mb1, mW2, mb2,
           fcW, fcb):
    src = edge_index[0].astype(jnp.int32)
    dst = edge_index[1].astype(jnp.int32)

    lin1 = _tc_lin1(x, W1)
    c_edge = _tc_edge_mlp(
        edge_attr, mW1, mb1.reshape(1, H), mW2,
        fcW[2 * H:3 * H, :], mb2.reshape(1, H), fcb.reshape(1, 1),
    ).reshape(E)

    lin1_pad = jnp.pad(lin1, ((0, NPAD - N), (0, 0)))
    srcT = src.reshape(E // S, S)
    dstT = dst.reshape(E // S, S)
    zeros2d = jnp.zeros((NPAD, H), jnp.float32)

    pred = _sc_mega(srcT, dstT, src, dst, lin1_pad, zeros2d,
                    W2.reshape(H * H), b1, b2,
                    fcW[0:H, 0], fcW[H:2 * H, 0], c_edge)
    return pred


# speedup vs baseline: 23.5747x; 1.0313x over previous
import functools

import jax
import jax.numpy as jnp
from jax import lax
from jax.experimental import pallas as pl
from jax.experimental.pallas import tpu as pltpu
from jax.experimental.pallas import tpu_sc as plsc

N = 10000
E = 320000
DF = 128
H = 16

NC, NS, L = 2, 16, 16
NW = NC * NS
NPAD = 10240
NPT = NPAD // NS
EPC = E
EPT = EPC // NS
CH = 2000
S = 125
KR = 8
RPT = EPT // S
NGRP = RPT // KR
EOUT_PT = E // NW
NOCH = EOUT_PT // CH


def _mesh():
    return plsc.VectorSubcoreMesh(core_axis_name="c", subcore_axis_name="s")


_SC_PARAMS = pltpu.CompilerParams(needs_layout_passes=False,
                                  use_tc_tiling_on_sc=False)


def _tc_lin1(x, W1):
    def body(x_ref, w_ref, o_ref):
        o_ref[...] = jnp.dot(x_ref[...], w_ref[...],
                             preferred_element_type=jnp.float32)
    return pl.pallas_call(
        body,
        out_shape=jax.ShapeDtypeStruct((N, H), jnp.float32),
    )(x, W1)


def _tc_edge_mlp(ea, mW1, mb1, mW2, f3, mb2, fcb):
    EB = 16000

    def body(ea_ref, w1_ref, b1_ref, w2_ref, f3_ref, b2_ref, fcb_ref, o_ref):
        t = jnp.dot(ea_ref[...], w1_ref[...],
                    preferred_element_type=jnp.float32) + b1_ref[...]
        t = jnp.maximum(t, 0.0)
        w3 = jnp.dot(w2_ref[...], f3_ref[...],
                     preferred_element_type=jnp.float32)
        cst = jnp.dot(b2_ref[...], f3_ref[...],
                      preferred_element_type=jnp.float32) + fcb_ref[...]
        o_ref[...] = (jnp.sum(t * w3.reshape(1, H), axis=1, keepdims=True)
                      + cst)

    return pl.pallas_call(
        body,
        grid=(E // EB,),
        in_specs=[
            pl.BlockSpec((EB, H), lambda i: (i, 0)),
            pl.BlockSpec((H, H), lambda i: (0, 0)),
            pl.BlockSpec((1, H), lambda i: (0, 0)),
            pl.BlockSpec((H, H), lambda i: (0, 0)),
            pl.BlockSpec((H, 1), lambda i: (0, 0)),
            pl.BlockSpec((1, H), lambda i: (0, 0)),
            pl.BlockSpec((1, 1), lambda i: (0, 0)),
        ],
        out_specs=pl.BlockSpec((EB, 1), lambda i: (i, 0)),
        out_shape=jax.ShapeDtypeStruct((E, 1), jnp.float32),
    )(ea, mW1, mb1, mW2, f3, mb2, fcb)


@functools.partial(
    pl.kernel,
    out_type=jax.ShapeDtypeStruct((E,), jnp.float32),
    mesh=_mesh(),
    compiler_params=_SC_PARAMS,
    scratch_types=[
        pltpu.VMEM((CH,), jnp.int32),
        pltpu.VMEM((CH,), jnp.int32),
        pltpu.VMEM((CH,), jnp.float32),
        pltpu.VMEM((CH,), jnp.float32),
        pltpu.VMEM((KR, S), jnp.int32),
        pltpu.VMEM((KR, S), jnp.int32),
        pltpu.VMEM((KR, S, H), jnp.float32),
        pltpu.VMEM((NPT, H), jnp.float32),
        pltpu.VMEM((NPT, H), jnp.float32),
        pltpu.VMEM((NPT, H), jnp.float32),
        pltpu.VMEM((NPT,), jnp.float32),
        pltpu.VMEM((NPT,), jnp.float32),
        pltpu.VMEM((NPT,), jnp.float32),
        pltpu.VMEM((NPT,), jnp.float32),
        pltpu.VMEM((H * H,), jnp.float32),
        pltpu.VMEM((H,), jnp.float32),
        pltpu.VMEM((H,), jnp.float32),
        pltpu.VMEM((H,), jnp.float32),
        pltpu.VMEM((H,), jnp.float32),
        pltpu.VMEM((NPAD,), jnp.float32),
        pltpu.VMEM((NPAD,), jnp.float32),
        pltpu.VMEM_SHARED((NS * NPAD,), jnp.float32),
        pltpu.VMEM_SHARED((NPAD, H), jnp.float32),
        pltpu.VMEM_SHARED((NPAD, H), jnp.float32),
        pltpu.VMEM_SHARED((NPAD, H), jnp.float32),
        pltpu.VMEM_SHARED((NPAD,), jnp.float32),
        pltpu.VMEM_SHARED((NPAD,), jnp.float32),
        pltpu.SemaphoreType.DMA,
        pltpu.SemaphoreType.DMA,
    ],
)
def _sc_mega(srcT_hbm, dstT_hbm, src_hbm, dst_hbm, lin1_hbm,
             w2_hbm, b1_hbm, b2_hbm, f1_hbm, f2_hbm, c_hbm, pred_hbm,
             idx_v, didx1_v, c_v, p_v, sidx_v, didx_v, rows_v,
             nbuf_v, g1s_v, g2s_v, dinv_v, tmp_v, av_v, bv_v,
             w2_v, b1_v, b2_v, f1_v, f2_v, a_full, b_full,
             part_sh, g1_sh, acc1_sh, g2_sh, a_sh, b_sh,
             gsem, ssem):
    cc = lax.axis_index("c")
    ss = lax.axis_index("s")
    wid = ss * NC + cc
    nbase = ss * NPT
    zeros = jnp.zeros((L,), jnp.float32)
    ones = jnp.ones((L,), jnp.float32)

    pltpu.sync_copy(w2_hbm, w2_v)
    pltpu.sync_copy(b1_hbm, b1_v)
    pltpu.sync_copy(b2_hbm, b2_v)
    pltpu.sync_copy(f1_hbm, f1_v)
    pltpu.sync_copy(f2_hbm, f2_v)

    zrow = jnp.zeros((L,), jnp.float32)

    def zn(n, c):
        nbuf_v[n] = zrow
        return c
    lax.fori_loop(0, NPT, zn, 0)
    pltpu.sync_copy(nbuf_v, acc1_sh.at[pl.ds(nbase, NPT)])

    def zb(i, c):
        a_full[pl.ds(i * L, L)] = zeros
        return c
    lax.fori_loop(0, NPAD // L, zb, 0)

    def dchunk(ci, c):
        pltpu.sync_copy(dst_hbm.at[pl.ds(ss * EPT + ci * CH, CH)], idx_v)

        def scat(j, c2):
            plsc.addupdate_scatter(a_full, [idx_v[pl.ds(j * L, L)]], ones)
            return c2
        lax.fori_loop(0, CH // L, scat, 0)
        return c
    lax.fori_loop(0, EPT // CH, dchunk, 0)
    pltpu.sync_copy(a_full, part_sh.at[pl.ds(ss * NPAD, NPAD)])
    plsc.subcore_barrier()

    def z2(i, c):
        dinv_v[pl.ds(i * L, L)] = zeros
        return c
    lax.fori_loop(0, NPT // L, z2, 0)
    for t in range(NS):
        pltpu.sync_copy(part_sh.at[pl.ds(t * NPAD + nbase, NPT)], tmp_v)

        def ab(i, c):
            sl = pl.ds(i * L, L)
            dinv_v[sl] = dinv_v[sl] + tmp_v[sl]
            return c
        lax.fori_loop(0, NPT // L, ab, 0)

    def newton(i, c):
        sl = pl.ds(i * L, L)
        xv = dinv_v[sl] + 1.0
        iv = plsc.bitcast(xv, jnp.int32)
        iv = 0x5F3759DF - lax.shift_right_logical(iv, 1)
        y = plsc.bitcast(iv, jnp.float32)
        y = y * (1.5 - 0.5 * xv * y * y)
        y = y * (1.5 - 0.5 * xv * y * y)
        y = y * (1.5 - 0.5 * xv * y * y)
        dinv_v[sl] = y
        return c
    lax.fori_loop(0, NPT // L, newton, 0)

    pltpu.sync_copy(lin1_hbm.at[pl.ds(nbase, NPT)], g1s_v)

    def g1b(g, c):
        dvec = dinv_v[pl.ds(g * L, L)]
        for t in range(L):
            n = g * L + t
            g1s_v[n] = g1s_v[n] * dvec[t]
        return c
    lax.fori_loop(0, NPT // L, g1b, 0)
    pltpu.sync_copy(g1s_v, g1_sh.at[pl.ds(nbase, NPT)])
    plsc.subcore_barrier()

    def aggregate(g_sh, acc_sh):
        def grp(ci, cr):
            rb = ss * RPT + ci * KR
            pltpu.sync_copy(srcT_hbm.at[pl.ds(rb, KR)], sidx_v)
            pltpu.sync_copy(dstT_hbm.at[pl.ds(rb, KR)], didx_v)
            hs = []
            for j in range(KR):
                hs.append(pltpu.async_copy(
                    g_sh.at[sidx_v.at[j]], rows_v.at[j], gsem))
            sc = []
            for j in range(KR):
                hs[j].wait()
                sc.append(pltpu.async_copy(
                    rows_v.at[j], acc_sh.at[didx_v.at[j]], ssem, add=True))
            for j in range(KR):
                sc[j].wait()
            return cr
        lax.fori_loop(0, NGRP, grp, 0)

    aggregate(g1_sh, acc1_sh)
    plsc.subcore_barrier()

    pltpu.sync_copy(acc1_sh.at[pl.ds(nbase, NPT)], nbuf_v)
    b1vec = b1_v[...]
    w2rows = [w2_v[pl.ds(k * H, H)] for k in range(H)]

    def s4b(g, c):
        dvec = dinv_v[pl.ds(g * L, L)]
        for t in range(L):
            n = g * L + t
            s = dvec[t]
            acc = nbuf_v[n] + g1s_v[n]
            h1 = jnp.maximum(s * acc + b1vec, 0.0)
            lin2 = h1[0] * w2rows[0]
            for k in range(1, H):
                lin2 = lin2 + h1[k] * w2rows[k]
            g2s_v[n] = s * lin2
        return c
    lax.fori_loop(0, NPT // L, s4b, 0)
    pltpu.sync_copy(g2s_v, g2_sh.at[pl.ds(nbase, NPT)])
    lax.fori_loop(0, NPT, zn, 0)
    pltpu.sync_copy(nbuf_v, acc1_sh.at[pl.ds(nbase, NPT)])
    plsc.subcore_barrier()

    aggregate(g2_sh, acc1_sh)
    plsc.subcore_barrier()

    pltpu.sync_copy(acc1_sh.at[pl.ds(nbase, NPT)], nbuf_v)
    b2vec = b2_v[...]
    f1vec = f1_v[...]
    f2vec = f2_v[...]
    lanes = lax.iota(jnp.int32, L)

    def s6b(g, c):
        dvec = dinv_v[pl.ds(g * L, L)]
        a_acc = jnp.zeros((L,), jnp.float32)
        b_acc = jnp.zeros((L,), jnp.float32)
        for t in range(L):
            n = g * L + t
            h2 = dvec[t] * (nbuf_v[n] + g2s_v[n]) + b2vec
            a_acc = jnp.where(lanes == t, jnp.sum(h2 * f1vec), a_acc)
            b_acc = jnp.where(lanes == t, jnp.sum(h2 * f2vec), b_acc)
        sl = pl.ds(g * L, L)
        av_v[sl] = a_acc
        bv_v[sl] = b_acc
        return c
    lax.fori_loop(0, NPT // L, s6b, 0)
    pltpu.sync_copy(av_v, a_sh.at[pl.ds(nbase, NPT)])
    pltpu.sync_copy(bv_v, b_sh.at[pl.ds(nbase, NPT)])
    plsc.subcore_barrier()

    pltpu.sync_copy(a_sh, a_full)
    pltpu.sync_copy(b_sh, b_full)
    ebase = wid * EOUT_PT

    def ochunk(ci, cr):
        base = ebase + ci * CH
        pltpu.sync_copy(src_hbm.at[pl.ds(base, CH)], idx_v)
        pltpu.sync_copy(dst_hbm.at[pl.ds(base, CH)], didx1_v)
        pltpu.sync_copy(c_hbm.at[pl.ds(base, CH)], c_v)

        def jb(j, c2):
            sl = pl.ds(j * L, L)
            av = plsc.load_gather(a_full, [idx_v[sl]])
            bv = plsc.load_gather(b_full, [didx1_v[sl]])
            p_v[sl] = av + bv + c_v[sl]
            return c2
        lax.fori_loop(0, CH // L, jb, 0)
        pltpu.sync_copy(p_v, pred_hbm.at[pl.ds(base, CH)])
        return cr
    lax.fori_loop(0, NOCH, ochunk, 0)


def kernel(x, edge_index, edge_attr, W1, b1, W2, b2, mW1, mb1, mW2, mb2,
           fcW, fcb):
    src = edge_index[0].astype(jnp.int32)
    dst = edge_index[1].astype(jnp.int32)

    lin1 = _tc_lin1(x, W1)
    c_edge = _tc_edge_mlp(
        edge_attr, mW1, mb1.reshape(1, H), mW2,
        fcW[2 * H:3 * H, :], mb2.reshape(1, H), fcb.reshape(1, 1),
    ).reshape(E)

    lin1_pad = jnp.pad(lin1, ((0, NPAD - N), (0, 0)))
    srcT = src.reshape(E // S, S)
    dstT = dst.reshape(E // S, S)

    pred = _sc_mega(srcT, dstT, src, dst, lin1_pad,
                    W2.reshape(H * H), b1, b2,
                    fcW[0:H, 0], fcW[H:2 * H, 0], c_edge)
    return pred


# speedup vs baseline: 30.3009x; 1.2853x over previous
import functools

import jax
import jax.numpy as jnp
from jax import lax
from jax.experimental import pallas as pl
from jax.experimental.pallas import tpu as pltpu
from jax.experimental.pallas import tpu_sc as plsc

N = 10000
E = 320000
DF = 128
H = 16

NC, NS, L = 2, 16, 16
NW = NC * NS
NPAD = 10240
NPT = NPAD // NS
EPC = E
EPT = EPC // NS
CH = 2000
S = 125
KR = 8
RPT = EPT // S
NGRP = RPT // KR
EOUT_PT = E // NW
NOCH = EOUT_PT // CH


def _mesh():
    return plsc.VectorSubcoreMesh(core_axis_name="c", subcore_axis_name="s")


_SC_PARAMS = pltpu.CompilerParams(needs_layout_passes=False,
                                  use_tc_tiling_on_sc=False)


def _tc_lin1(x, W1):
    def body(x_ref, w_ref, o_ref):
        o_ref[...] = jnp.dot(x_ref[...], w_ref[...],
                             preferred_element_type=jnp.float32)
    return pl.pallas_call(
        body,
        out_shape=jax.ShapeDtypeStruct((N, H), jnp.float32),
    )(x, W1)


def _tc_edge_mlp(ea8, W1big, b8, mW2, f3, mb2, fcb):
    R = E // 8
    RB = 8000

    def body(ea_ref, w1_ref, b8_ref, w2_ref, f3_ref, b2_ref, fcb_ref, o_ref):
        t8 = jnp.dot(ea_ref[...], w1_ref[...],
                     preferred_element_type=jnp.float32) + b8_ref[...]
        t8 = jnp.maximum(t8, 0.0)
        w3 = jnp.dot(w2_ref[...], f3_ref[...],
                     preferred_element_type=jnp.float32)
        w3t = jnp.concatenate([w3] * 8, axis=0)
        r0 = lax.broadcasted_iota(jnp.int32, (128, 8), 0)
        r1 = lax.broadcasted_iota(jnp.int32, (128, 8), 1)
        wsel = jnp.where(r0 // H == r1, w3t, 0.0)
        cst = jnp.dot(b2_ref[...], f3_ref[...],
                      preferred_element_type=jnp.float32) + fcb_ref[...]
        o_ref[...] = jnp.dot(t8, wsel,
                             preferred_element_type=jnp.float32) + cst

    return pl.pallas_call(
        body,
        grid=(R // RB,),
        in_specs=[
            pl.BlockSpec((RB, 128), lambda i: (i, 0)),
            pl.BlockSpec((128, 128), lambda i: (0, 0)),
            pl.BlockSpec((1, 128), lambda i: (0, 0)),
            pl.BlockSpec((H, H), lambda i: (0, 0)),
            pl.BlockSpec((H, 1), lambda i: (0, 0)),
            pl.BlockSpec((1, H), lambda i: (0, 0)),
            pl.BlockSpec((1, 1), lambda i: (0, 0)),
        ],
        out_specs=pl.BlockSpec((RB, 8), lambda i: (i, 0)),
        out_shape=jax.ShapeDtypeStruct((R, 8), jnp.float32),
    )(ea8, W1big, b8, mW2, f3, mb2, fcb)


@functools.partial(
    pl.kernel,
    out_type=jax.ShapeDtypeStruct((E,), jnp.float32),
    mesh=_mesh(),
    compiler_params=_SC_PARAMS,
    scratch_types=[
        pltpu.VMEM((CH,), jnp.int32),
        pltpu.VMEM((CH,), jnp.int32),
        pltpu.VMEM((CH,), jnp.float32),
        pltpu.VMEM((CH,), jnp.float32),
        pltpu.VMEM((KR, S), jnp.int32),
        pltpu.VMEM((KR, S), jnp.int32),
        pltpu.VMEM((KR, S, H), jnp.float32),
        pltpu.VMEM((NPT, H), jnp.float32),
        pltpu.VMEM((NPT, H), jnp.float32),
        pltpu.VMEM((NPT, H), jnp.float32),
        pltpu.VMEM((NPT,), jnp.float32),
        pltpu.VMEM((NPT,), jnp.float32),
        pltpu.VMEM((NPT,), jnp.float32),
        pltpu.VMEM((NPT,), jnp.float32),
        pltpu.VMEM((H * H,), jnp.float32),
        pltpu.VMEM((H,), jnp.float32),
        pltpu.VMEM((H,), jnp.float32),
        pltpu.VMEM((H,), jnp.float32),
        pltpu.VMEM((H,), jnp.float32),
        pltpu.VMEM((NPAD,), jnp.float32),
        pltpu.VMEM((NPAD,), jnp.float32),
        pltpu.VMEM_SHARED((NS * NPAD,), jnp.float32),
        pltpu.VMEM_SHARED((NPAD, H), jnp.float32),
        pltpu.VMEM_SHARED((NPAD, H), jnp.float32),
        pltpu.VMEM_SHARED((NPAD, H), jnp.float32),
        pltpu.VMEM_SHARED((NPAD,), jnp.float32),
        pltpu.VMEM_SHARED((NPAD,), jnp.float32),
        pltpu.SemaphoreType.DMA,
        pltpu.SemaphoreType.DMA,
    ],
)
def _sc_mega(srcT_hbm, dstT_hbm, src_hbm, dst_hbm, lin1_hbm,
             w2_hbm, b1_hbm, b2_hbm, f1_hbm, f2_hbm, c_hbm, pred_hbm,
             idx_v, didx1_v, c_v, p_v, sidx_v, didx_v, rows_v,
             nbuf_v, g1s_v, g2s_v, dinv_v, tmp_v, av_v, bv_v,
             w2_v, b1_v, b2_v, f1_v, f2_v, a_full, b_full,
             part_sh, g1_sh, acc1_sh, g2_sh, a_sh, b_sh,
             gsem, ssem):
    cc = lax.axis_index("c")
    ss = lax.axis_index("s")
    wid = ss * NC + cc
    nbase = ss * NPT
    zeros = jnp.zeros((L,), jnp.float32)
    ones = jnp.ones((L,), jnp.float32)

    pltpu.sync_copy(w2_hbm, w2_v)
    pltpu.sync_copy(b1_hbm, b1_v)
    pltpu.sync_copy(b2_hbm, b2_v)
    pltpu.sync_copy(f1_hbm, f1_v)
    pltpu.sync_copy(f2_hbm, f2_v)

    zrow = jnp.zeros((L,), jnp.float32)

    def zn(n, c):
        nbuf_v[n] = zrow
        return c
    lax.fori_loop(0, NPT, zn, 0)
    pltpu.sync_copy(nbuf_v, acc1_sh.at[pl.ds(nbase, NPT)])

    def zb(i, c):
        a_full[pl.ds(i * L, L)] = zeros
        return c
    lax.fori_loop(0, NPAD // L, zb, 0)

    def dchunk(ci, c):
        pltpu.sync_copy(dst_hbm.at[pl.ds(ss * EPT + ci * CH, CH)], idx_v)

        def scat(j, c2):
            plsc.addupdate_scatter(a_full, [idx_v[pl.ds(j * L, L)]], ones)
            return c2
        lax.fori_loop(0, CH // L, scat, 0)
        return c
    lax.fori_loop(0, EPT // CH, dchunk, 0)
    pltpu.sync_copy(a_full, part_sh.at[pl.ds(ss * NPAD, NPAD)])
    plsc.subcore_barrier()

    def z2(i, c):
        dinv_v[pl.ds(i * L, L)] = zeros
        return c
    lax.fori_loop(0, NPT // L, z2, 0)
    for t in range(NS):
        pltpu.sync_copy(part_sh.at[pl.ds(t * NPAD + nbase, NPT)], tmp_v)

        def ab(i, c):
            sl = pl.ds(i * L, L)
            dinv_v[sl] = dinv_v[sl] + tmp_v[sl]
            return c
        lax.fori_loop(0, NPT // L, ab, 0)

    def newton(i, c):
        sl = pl.ds(i * L, L)
        xv = dinv_v[sl] + 1.0
        iv = plsc.bitcast(xv, jnp.int32)
        iv = 0x5F3759DF - lax.shift_right_logical(iv, 1)
        y = plsc.bitcast(iv, jnp.float32)
        y = y * (1.5 - 0.5 * xv * y * y)
        y = y * (1.5 - 0.5 * xv * y * y)
        y = y * (1.5 - 0.5 * xv * y * y)
        dinv_v[sl] = y
        return c
    lax.fori_loop(0, NPT // L, newton, 0)

    pltpu.sync_copy(lin1_hbm.at[pl.ds(nbase, NPT)], g1s_v)

    def g1b(g, c):
        dvec = dinv_v[pl.ds(g * L, L)]
        for t in range(L):
            n = g * L + t
            g1s_v[n] = g1s_v[n] * dvec[t]
        return c
    lax.fori_loop(0, NPT // L, g1b, 0)
    pltpu.sync_copy(g1s_v, g1_sh.at[pl.ds(nbase, NPT)])
    plsc.subcore_barrier()

    def aggregate(g_sh, acc_sh):
        def grp(ci, cr):
            rb = ss * RPT + ci * KR
            pltpu.sync_copy(srcT_hbm.at[pl.ds(rb, KR)], sidx_v)
            pltpu.sync_copy(dstT_hbm.at[pl.ds(rb, KR)], didx_v)
            hs = []
            for j in range(KR):
                hs.append(pltpu.async_copy(
                    g_sh.at[sidx_v.at[j]], rows_v.at[j], gsem))
            sc = []
            for j in range(KR):
                hs[j].wait()
                sc.append(pltpu.async_copy(
                    rows_v.at[j], acc_sh.at[didx_v.at[j]], ssem, add=True))
            for j in range(KR):
                sc[j].wait()
            return cr
        lax.fori_loop(0, NGRP, grp, 0)

    aggregate(g1_sh, acc1_sh)
    plsc.subcore_barrier()

    pltpu.sync_copy(acc1_sh.at[pl.ds(nbase, NPT)], nbuf_v)
    b1vec = b1_v[...]
    w2rows = [w2_v[pl.ds(k * H, H)] for k in range(H)]

    def s4b(g, c):
        dvec = dinv_v[pl.ds(g * L, L)]
        for t in range(L):
            n = g * L + t
            s = dvec[t]
            acc = nbuf_v[n] + g1s_v[n]
            h1 = jnp.maximum(s * acc + b1vec, 0.0)
            lin2 = h1[0] * w2rows[0]
            for k in range(1, H):
                lin2 = lin2 + h1[k] * w2rows[k]
            g2s_v[n] = s * lin2
        return c
    lax.fori_loop(0, NPT // L, s4b, 0)
    pltpu.sync_copy(g2s_v, g2_sh.at[pl.ds(nbase, NPT)])
    lax.fori_loop(0, NPT, zn, 0)
    pltpu.sync_copy(nbuf_v, acc1_sh.at[pl.ds(nbase, NPT)])
    plsc.subcore_barrier()

    aggregate(g2_sh, acc1_sh)
    plsc.subcore_barrier()

    pltpu.sync_copy(acc1_sh.at[pl.ds(nbase, NPT)], nbuf_v)
    b2vec = b2_v[...]
    f1vec = f1_v[...]
    f2vec = f2_v[...]
    lanes = lax.iota(jnp.int32, L)

    def s6b(g, c):
        dvec = dinv_v[pl.ds(g * L, L)]
        a_acc = jnp.zeros((L,), jnp.float32)
        b_acc = jnp.zeros((L,), jnp.float32)
        for t in range(L):
            n = g * L + t
            h2 = dvec[t] * (nbuf_v[n] + g2s_v[n]) + b2vec
            a_acc = jnp.where(lanes == t, jnp.sum(h2 * f1vec), a_acc)
            b_acc = jnp.where(lanes == t, jnp.sum(h2 * f2vec), b_acc)
        sl = pl.ds(g * L, L)
        av_v[sl] = a_acc
        bv_v[sl] = b_acc
        return c
    lax.fori_loop(0, NPT // L, s6b, 0)
    pltpu.sync_copy(av_v, a_sh.at[pl.ds(nbase, NPT)])
    pltpu.sync_copy(bv_v, b_sh.at[pl.ds(nbase, NPT)])
    plsc.subcore_barrier()

    pltpu.sync_copy(a_sh, a_full)
    pltpu.sync_copy(b_sh, b_full)
    ebase = wid * EOUT_PT

    def ochunk(ci, cr):
        base = ebase + ci * CH
        pltpu.sync_copy(src_hbm.at[pl.ds(base, CH)], idx_v)
        pltpu.sync_copy(dst_hbm.at[pl.ds(base, CH)], didx1_v)
        pltpu.sync_copy(c_hbm.at[pl.ds(base, CH)], c_v)

        def jb(j, c2):
            sl = pl.ds(j * L, L)
            av = plsc.load_gather(a_full, [idx_v[sl]])
            bv = plsc.load_gather(b_full, [didx1_v[sl]])
            p_v[sl] = av + bv + c_v[sl]
            return c2
        lax.fori_loop(0, CH // L, jb, 0)
        pltpu.sync_copy(p_v, pred_hbm.at[pl.ds(base, CH)])
        return cr
    lax.fori_loop(0, NOCH, ochunk, 0)


def kernel(x, edge_index, edge_attr, W1, b1, W2, b2, mW1, mb1, mW2, mb2,
           fcW, fcb):
    src = edge_index[0].astype(jnp.int32)
    dst = edge_index[1].astype(jnp.int32)

    lin1 = _tc_lin1(x, W1)
    eye8 = jnp.eye(8, dtype=jnp.float32)
    c_edge = _tc_edge_mlp(
        edge_attr.reshape(E // 8, 128), jnp.kron(eye8, mW1),
        jnp.tile(mb1, 8).reshape(1, 128), mW2,
        fcW[2 * H:3 * H, :], mb2.reshape(1, H), fcb.reshape(1, 1),
    ).reshape(E)

    lin1_pad = jnp.pad(lin1, ((0, NPAD - N), (0, 0)))
    srcT = src.reshape(E // S, S)
    dstT = dst.reshape(E // S, S)

    pred = _sc_mega(srcT, dstT, src, dst, lin1_pad,
                    W2.reshape(H * H), b1, b2,
                    fcW[0:H, 0], fcW[H:2 * H, 0], c_edge)
    return pred


# speedup vs baseline: 30.5799x; 1.0092x over previous
import functools

import jax
import jax.numpy as jnp
from jax import lax
from jax.experimental import pallas as pl
from jax.experimental.pallas import tpu as pltpu
from jax.experimental.pallas import tpu_sc as plsc

N = 10000
E = 320000
DF = 128
H = 16

NC, NS, L = 2, 16, 16
NW = NC * NS
NPAD = 10240
NPT = NPAD // NS
EPC = E
EPT = EPC // NS
CH = 2000
S = 125
KR = 8
RPT = EPT // S
NGRP = RPT // KR
EOUT_PT = E // NW
NOCH = EOUT_PT // CH


def _mesh():
    return plsc.VectorSubcoreMesh(core_axis_name="c", subcore_axis_name="s")


_SC_PARAMS = pltpu.CompilerParams(needs_layout_passes=False,
                                  use_tc_tiling_on_sc=False)


def _tc_lin1(x, W1):
    def body(x_ref, w_ref, o_ref):
        o_ref[...] = jnp.zeros((NPAD, H), jnp.float32)
        o_ref[pl.ds(0, N), :] = jnp.dot(x_ref[...], w_ref[...],
                                        preferred_element_type=jnp.float32)
    return pl.pallas_call(
        body,
        out_shape=jax.ShapeDtypeStruct((NPAD, H), jnp.float32),
    )(x, W1)


def _tc_edge_mlp(ea8, W1big, b8, mW2, f3, mb2, fcb):
    R = E // 8
    RB = 8000

    def body(ea_ref, w1_ref, b8_ref, w2_ref, f3_ref, b2_ref, fcb_ref, o_ref):
        t8 = jnp.dot(ea_ref[...], w1_ref[...],
                     preferred_element_type=jnp.float32) + b8_ref[...]
        t8 = jnp.maximum(t8, 0.0)
        w3 = jnp.dot(w2_ref[...], f3_ref[...],
                     preferred_element_type=jnp.float32)
        w3t = jnp.concatenate([w3] * 8, axis=0)
        r0 = lax.broadcasted_iota(jnp.int32, (128, 8), 0)
        r1 = lax.broadcasted_iota(jnp.int32, (128, 8), 1)
        wsel = jnp.where(r0 // H == r1, w3t, 0.0)
        cst = jnp.dot(b2_ref[...], f3_ref[...],
                      preferred_element_type=jnp.float32) + fcb_ref[...]
        o_ref[...] = jnp.dot(t8, wsel,
                             preferred_element_type=jnp.float32) + cst

    return pl.pallas_call(
        body,
        grid=(R // RB,),
        in_specs=[
            pl.BlockSpec((RB, 128), lambda i: (i, 0)),
            pl.BlockSpec((128, 128), lambda i: (0, 0)),
            pl.BlockSpec((1, 128), lambda i: (0, 0)),
            pl.BlockSpec((H, H), lambda i: (0, 0)),
            pl.BlockSpec((H, 1), lambda i: (0, 0)),
            pl.BlockSpec((1, H), lambda i: (0, 0)),
            pl.BlockSpec((1, 1), lambda i: (0, 0)),
        ],
        out_specs=pl.BlockSpec((RB, 8), lambda i: (i, 0)),
        out_shape=jax.ShapeDtypeStruct((R, 8), jnp.float32),
    )(ea8, W1big, b8, mW2, f3, mb2, fcb)


@functools.partial(
    pl.kernel,
    out_type=jax.ShapeDtypeStruct((E,), jnp.float32),
    mesh=_mesh(),
    compiler_params=_SC_PARAMS,
    scratch_types=[
        pltpu.VMEM((CH,), jnp.int32),
        pltpu.VMEM((CH,), jnp.int32),
        pltpu.VMEM((CH,), jnp.float32),
        pltpu.VMEM((CH,), jnp.float32),
        pltpu.VMEM((KR, S), jnp.int32),
        pltpu.VMEM((KR, S), jnp.int32),
        pltpu.VMEM((KR, S, H), jnp.float32),
        pltpu.VMEM((NPT, H), jnp.float32),
        pltpu.VMEM((NPT, H), jnp.float32),
        pltpu.VMEM((NPT, H), jnp.float32),
        pltpu.VMEM((NPT,), jnp.float32),
        pltpu.VMEM((NPT,), jnp.float32),
        pltpu.VMEM((NPT,), jnp.float32),
        pltpu.VMEM((NPT,), jnp.float32),
        pltpu.VMEM((H * H,), jnp.float32),
        pltpu.VMEM((H,), jnp.float32),
        pltpu.VMEM((H,), jnp.float32),
        pltpu.VMEM((H,), jnp.float32),
        pltpu.VMEM((H,), jnp.float32),
        pltpu.VMEM((NPAD,), jnp.float32),
        pltpu.VMEM((NPAD,), jnp.float32),
        pltpu.VMEM_SHARED((NS * NPAD,), jnp.float32),
        pltpu.VMEM_SHARED((NPAD, H), jnp.float32),
        pltpu.VMEM_SHARED((NPAD, H), jnp.float32),
        pltpu.VMEM_SHARED((NPAD, H), jnp.float32),
        pltpu.VMEM_SHARED((NPAD,), jnp.float32),
        pltpu.VMEM_SHARED((NPAD,), jnp.float32),
        pltpu.SemaphoreType.DMA,
        pltpu.SemaphoreType.DMA,
    ],
)
def _sc_mega(eiT_hbm, ei_hbm, lin1_hbm,
             w2_hbm, b1_hbm, b2_hbm, f1_hbm, f2_hbm, c_hbm, pred_hbm,
             idx_v, didx1_v, c_v, p_v, sidx_v, didx_v, rows_v,
             nbuf_v, g1s_v, g2s_v, dinv_v, tmp_v, av_v, bv_v,
             w2_v, b1_v, b2_v, f1_v, f2_v, a_full, b_full,
             part_sh, g1_sh, acc1_sh, g2_sh, a_sh, b_sh,
             gsem, ssem):
    cc = lax.axis_index("c")
    ss = lax.axis_index("s")
    wid = ss * NC + cc
    nbase = ss * NPT
    zeros = jnp.zeros((L,), jnp.float32)
    ones = jnp.ones((L,), jnp.float32)

    pltpu.sync_copy(w2_hbm, w2_v)
    pltpu.sync_copy(b1_hbm, b1_v)
    pltpu.sync_copy(b2_hbm, b2_v)
    pltpu.sync_copy(f1_hbm, f1_v)
    pltpu.sync_copy(f2_hbm, f2_v)

    zrow = jnp.zeros((L,), jnp.float32)

    def zn(n, c):
        nbuf_v[n] = zrow
        return c
    lax.fori_loop(0, NPT, zn, 0)
    pltpu.sync_copy(nbuf_v, acc1_sh.at[pl.ds(nbase, NPT)])

    def zb(i, c):
        a_full[pl.ds(i * L, L)] = zeros
        return c
    lax.fori_loop(0, NPAD // L, zb, 0)

    def dchunk(ci, c):
        pltpu.sync_copy(ei_hbm.at[1, pl.ds(ss * EPT + ci * CH, CH)], idx_v)

        def scat(j, c2):
            plsc.addupdate_scatter(a_full, [idx_v[pl.ds(j * L, L)]], ones)
            return c2
        lax.fori_loop(0, CH // L, scat, 0)
        return c
    lax.fori_loop(0, EPT // CH, dchunk, 0)
    pltpu.sync_copy(a_full, part_sh.at[pl.ds(ss * NPAD, NPAD)])
    plsc.subcore_barrier()

    def z2(i, c):
        dinv_v[pl.ds(i * L, L)] = zeros
        return c
    lax.fori_loop(0, NPT // L, z2, 0)
    for t in range(NS):
        pltpu.sync_copy(part_sh.at[pl.ds(t * NPAD + nbase, NPT)], tmp_v)

        def ab(i, c):
            sl = pl.ds(i * L, L)
            dinv_v[sl] = dinv_v[sl] + tmp_v[sl]
            return c
        lax.fori_loop(0, NPT // L, ab, 0)

    def newton(i, c):
        sl = pl.ds(i * L, L)
        xv = dinv_v[sl] + 1.0
        iv = plsc.bitcast(xv, jnp.int32)
        iv = 0x5F3759DF - lax.shift_right_logical(iv, 1)
        y = plsc.bitcast(iv, jnp.float32)
        y = y * (1.5 - 0.5 * xv * y * y)
        y = y * (1.5 - 0.5 * xv * y * y)
        y = y * (1.5 - 0.5 * xv * y * y)
        dinv_v[sl] = y
        return c
    lax.fori_loop(0, NPT // L, newton, 0)

    pltpu.sync_copy(lin1_hbm.at[pl.ds(nbase, NPT)], g1s_v)

    def g1b(g, c):
        dvec = dinv_v[pl.ds(g * L, L)]
        for t in range(L):
            n = g * L + t
            g1s_v[n] = g1s_v[n] * dvec[t]
        return c
    lax.fori_loop(0, NPT // L, g1b, 0)
    pltpu.sync_copy(g1s_v, g1_sh.at[pl.ds(nbase, NPT)])
    plsc.subcore_barrier()

    def aggregate(g_sh, acc_sh):
        def grp(ci, cr):
            rb = ss * RPT + ci * KR
            pltpu.sync_copy(eiT_hbm.at[0, pl.ds(rb, KR)], sidx_v)
            pltpu.sync_copy(eiT_hbm.at[1, pl.ds(rb, KR)], didx_v)
            hs = []
            for j in range(KR):
                hs.append(pltpu.async_copy(
                    g_sh.at[sidx_v.at[j]], rows_v.at[j], gsem))
            sc = []
            for j in range(KR):
                hs[j].wait()
                sc.append(pltpu.async_copy(
                    rows_v.at[j], acc_sh.at[didx_v.at[j]], ssem, add=True))
            for j in range(KR):
                sc[j].wait()
            return cr
        lax.fori_loop(0, NGRP, grp, 0)

    aggregate(g1_sh, acc1_sh)
    plsc.subcore_barrier()

    pltpu.sync_copy(acc1_sh.at[pl.ds(nbase, NPT)], nbuf_v)
    b1vec = b1_v[...]
    w2rows = [w2_v[pl.ds(k * H, H)] for k in range(H)]

    def s4b(g, c):
        dvec = dinv_v[pl.ds(g * L, L)]
        for t in range(L):
            n = g * L + t
            s = dvec[t]
            acc = nbuf_v[n] + g1s_v[n]
            h1 = jnp.maximum(s * acc + b1vec, 0.0)
            lin2 = h1[0] * w2rows[0]
            for k in range(1, H):
                lin2 = lin2 + h1[k] * w2rows[k]
            g2s_v[n] = s * lin2
        return c
    lax.fori_loop(0, NPT // L, s4b, 0)
    pltpu.sync_copy(g2s_v, g2_sh.at[pl.ds(nbase, NPT)])
    lax.fori_loop(0, NPT, zn, 0)
    pltpu.sync_copy(nbuf_v, acc1_sh.at[pl.ds(nbase, NPT)])
    plsc.subcore_barrier()

    aggregate(g2_sh, acc1_sh)
    plsc.subcore_barrier()

    pltpu.sync_copy(acc1_sh.at[pl.ds(nbase, NPT)], nbuf_v)
    b2vec = b2_v[...]
    f1vec = f1_v[...]
    f2vec = f2_v[...]
    lanes = lax.iota(jnp.int32, L)

    def s6b(g, c):
        dvec = dinv_v[pl.ds(g * L, L)]
        a_acc = jnp.zeros((L,), jnp.float32)
        b_acc = jnp.zeros((L,), jnp.float32)
        for t in range(L):
            n = g * L + t
            h2 = dvec[t] * (nbuf_v[n] + g2s_v[n]) + b2vec
            a_acc = jnp.where(lanes == t, jnp.sum(h2 * f1vec), a_acc)
            b_acc = jnp.where(lanes == t, jnp.sum(h2 * f2vec), b_acc)
        sl = pl.ds(g * L, L)
        av_v[sl] = a_acc
        bv_v[sl] = b_acc
        return c
    lax.fori_loop(0, NPT // L, s6b, 0)
    pltpu.sync_copy(av_v, a_sh.at[pl.ds(nbase, NPT)])
    pltpu.sync_copy(bv_v, b_sh.at[pl.ds(nbase, NPT)])
    plsc.subcore_barrier()

    pltpu.sync_copy(a_sh, a_full)
    pltpu.sync_copy(b_sh, b_full)
    ebase = wid * EOUT_PT

    def ochunk(ci, cr):
        base = ebase + ci * CH
        pltpu.sync_copy(ei_hbm.at[0, pl.ds(base, CH)], idx_v)
        pltpu.sync_copy(ei_hbm.at[1, pl.ds(base, CH)], didx1_v)
        pltpu.sync_copy(c_hbm.at[pl.ds(base, CH)], c_v)

        def jb(j, c2):
            sl = pl.ds(j * L, L)
            av = plsc.load_gather(a_full, [idx_v[sl]])
            bv = plsc.load_gather(b_full, [didx1_v[sl]])
            p_v[sl] = av + bv + c_v[sl]
            return c2
        lax.fori_loop(0, CH // L, jb, 0)
        pltpu.sync_copy(p_v, pred_hbm.at[pl.ds(base, CH)])
        return cr
    lax.fori_loop(0, NOCH, ochunk, 0)


def kernel(x, edge_index, edge_attr, W1, b1, W2, b2, mW1, mb1, mW2, mb2,
           fcW, fcb):
    ei = edge_index.astype(jnp.int32)

    lin1_pad = _tc_lin1(x, W1)
    eye8 = jnp.eye(8, dtype=jnp.float32)
    c_edge = _tc_edge_mlp(
        edge_attr.reshape(E // 8, 128), jnp.kron(eye8, mW1),
        jnp.tile(mb1, 8).reshape(1, 128), mW2,
        fcW[2 * H:3 * H, :], mb2.reshape(1, H), fcb.reshape(1, 1),
    ).reshape(E)

    eiT = ei.reshape(2, E // S, S)

    pred = _sc_mega(eiT, ei, lin1_pad,
                    W2.reshape(H * H), b1, b2,
                    fcW[0:H, 0], fcW[H:2 * H, 0], c_edge)
    return pred


# speedup vs baseline: 30.8378x; 1.0084x over previous
import functools

import jax
import jax.numpy as jnp
from jax import lax
from jax.experimental import pallas as pl
from jax.experimental.pallas import tpu as pltpu
from jax.experimental.pallas import tpu_sc as plsc

N = 10000
E = 320000
DF = 128
H = 16

NC, NS, L = 2, 16, 16
NW = NC * NS
NPAD = 10240
NPT = NPAD // NS
EPC = E
EPT = EPC // NS
CH = 2000
S = 125
KR = 8
RPT = EPT // S
NGRP = RPT // KR
EOUT_PT = E // NW
NOCH = EOUT_PT // CH


def _mesh():
    return plsc.VectorSubcoreMesh(core_axis_name="c", subcore_axis_name="s")


_SC_PARAMS = pltpu.CompilerParams(needs_layout_passes=False,
                                  use_tc_tiling_on_sc=False)


def _tc_dense(x, W1, ea8, W1big, b8, mW2, f3, mb2, fcb):
    R = E // 8
    RB = 8000

    def body(x_ref, w1_ref, ea_ref, w1b_ref, b8_ref, w2_ref, f3_ref,
             b2_ref, fcb_ref, c_ref, lin_ref):
        t8 = jnp.dot(ea_ref[...], w1b_ref[...],
                     preferred_element_type=jnp.float32) + b8_ref[...]
        t8 = jnp.maximum(t8, 0.0)
        w3 = jnp.dot(w2_ref[...], f3_ref[...],
                     preferred_element_type=jnp.float32)
        w3t = jnp.concatenate([w3] * 8, axis=0)
        r0 = lax.broadcasted_iota(jnp.int32, (128, 8), 0)
        r1 = lax.broadcasted_iota(jnp.int32, (128, 8), 1)
        wsel = jnp.where(r0 // H == r1, w3t, 0.0)
        cst = jnp.dot(b2_ref[...], f3_ref[...],
                      preferred_element_type=jnp.float32) + fcb_ref[...]
        c_ref[...] = jnp.dot(t8, wsel,
                             preferred_element_type=jnp.float32) + cst

        @pl.when(pl.program_id(0) == 0)
        def _():
            lin_ref[...] = jnp.zeros((NPAD, H), jnp.float32)
            lin_ref[pl.ds(0, N), :] = jnp.dot(
                x_ref[...], w1_ref[...], preferred_element_type=jnp.float32)

    return pl.pallas_call(
        body,
        grid=(R // RB,),
        in_specs=[
            pl.BlockSpec((N, DF), lambda i: (0, 0)),
            pl.BlockSpec((DF, H), lambda i: (0, 0)),
            pl.BlockSpec((RB, 128), lambda i: (i, 0)),
            pl.BlockSpec((128, 128), lambda i: (0, 0)),
            pl.BlockSpec((1, 128), lambda i: (0, 0)),
            pl.BlockSpec((H, H), lambda i: (0, 0)),
            pl.BlockSpec((H, 1), lambda i: (0, 0)),
            pl.BlockSpec((1, H), lambda i: (0, 0)),
            pl.BlockSpec((1, 1), lambda i: (0, 0)),
        ],
        out_specs=[
            pl.BlockSpec((RB, 8), lambda i: (i, 0)),
            pl.BlockSpec((NPAD, H), lambda i: (0, 0)),
        ],
        out_shape=[
            jax.ShapeDtypeStruct((R, 8), jnp.float32),
            jax.ShapeDtypeStruct((NPAD, H), jnp.float32),
        ],
    )(x, W1, ea8, W1big, b8, mW2, f3, mb2, fcb)


@functools.partial(
    pl.kernel,
    out_type=jax.ShapeDtypeStruct((E,), jnp.float32),
    mesh=_mesh(),
    compiler_params=_SC_PARAMS,
    scratch_types=[
        pltpu.VMEM((CH,), jnp.int32),
        pltpu.VMEM((CH,), jnp.int32),
        pltpu.VMEM((CH,), jnp.float32),
        pltpu.VMEM((CH,), jnp.float32),
        pltpu.VMEM((KR, S), jnp.int32),
        pltpu.VMEM((KR, S), jnp.int32),
        pltpu.VMEM((KR, S, H), jnp.float32),
        pltpu.VMEM((NPT, H), jnp.float32),
        pltpu.VMEM((NPT, H), jnp.float32),
        pltpu.VMEM((NPT, H), jnp.float32),
        pltpu.VMEM((NPT,), jnp.float32),
        pltpu.VMEM((NPT,), jnp.float32),
        pltpu.VMEM((NPT,), jnp.float32),
        pltpu.VMEM((NPT,), jnp.float32),
        pltpu.VMEM((H * H,), jnp.float32),
        pltpu.VMEM((H,), jnp.float32),
        pltpu.VMEM((H,), jnp.float32),
        pltpu.VMEM((H,), jnp.float32),
        pltpu.VMEM((H,), jnp.float32),
        pltpu.VMEM((NPAD,), jnp.float32),
        pltpu.VMEM((NPAD,), jnp.float32),
        pltpu.VMEM_SHARED((NS * NPAD,), jnp.float32),
        pltpu.VMEM_SHARED((NPAD, H), jnp.float32),
        pltpu.VMEM_SHARED((NPAD, H), jnp.float32),
        pltpu.VMEM_SHARED((NPAD, H), jnp.float32),
        pltpu.VMEM_SHARED((NPAD,), jnp.float32),
        pltpu.VMEM_SHARED((NPAD,), jnp.float32),
        pltpu.SemaphoreType.DMA,
        pltpu.SemaphoreType.DMA,
    ],
)
def _sc_mega(eiT_hbm, ei_hbm, lin1_hbm,
             w2_hbm, b1_hbm, b2_hbm, f1_hbm, f2_hbm, c_hbm, pred_hbm,
             idx_v, didx1_v, c_v, p_v, sidx_v, didx_v, rows_v,
             nbuf_v, g1s_v, g2s_v, dinv_v, tmp_v, av_v, bv_v,
             w2_v, b1_v, b2_v, f1_v, f2_v, a_full, b_full,
             part_sh, g1_sh, acc1_sh, g2_sh, a_sh, b_sh,
             gsem, ssem):
    cc = lax.axis_index("c")
    ss = lax.axis_index("s")
    wid = ss * NC + cc
    nbase = ss * NPT
    zeros = jnp.zeros((L,), jnp.float32)
    ones = jnp.ones((L,), jnp.float32)

    pltpu.sync_copy(w2_hbm, w2_v)
    pltpu.sync_copy(b1_hbm, b1_v)
    pltpu.sync_copy(b2_hbm, b2_v)
    pltpu.sync_copy(f1_hbm, f1_v)
    pltpu.sync_copy(f2_hbm, f2_v)

    zrow = jnp.zeros((L,), jnp.float32)

    def zn(n, c):
        nbuf_v[n] = zrow
        return c
    lax.fori_loop(0, NPT, zn, 0)
    pltpu.sync_copy(nbuf_v, acc1_sh.at[pl.ds(nbase, NPT)])

    def zb(i, c):
        a_full[pl.ds(i * L, L)] = zeros
        return c
    lax.fori_loop(0, NPAD // L, zb, 0)

    def dchunk(ci, c):
        pltpu.sync_copy(ei_hbm.at[1, pl.ds(ss * EPT + ci * CH, CH)], idx_v)

        def scat(j, c2):
            plsc.addupdate_scatter(a_full, [idx_v[pl.ds(j * L, L)]], ones)
            return c2
        lax.fori_loop(0, CH // L, scat, 0)
        return c
    lax.fori_loop(0, EPT // CH, dchunk, 0)
    pltpu.sync_copy(a_full, part_sh.at[pl.ds(ss * NPAD, NPAD)])
    plsc.subcore_barrier()

    def z2(i, c):
        dinv_v[pl.ds(i * L, L)] = zeros
        return c
    lax.fori_loop(0, NPT // L, z2, 0)
    for t in range(NS):
        pltpu.sync_copy(part_sh.at[pl.ds(t * NPAD + nbase, NPT)], tmp_v)

        def ab(i, c):
            sl = pl.ds(i * L, L)
            dinv_v[sl] = dinv_v[sl] + tmp_v[sl]
            return c
        lax.fori_loop(0, NPT // L, ab, 0)

    def newton(i, c):
        sl = pl.ds(i * L, L)
        xv = dinv_v[sl] + 1.0
        iv = plsc.bitcast(xv, jnp.int32)
        iv = 0x5F3759DF - lax.shift_right_logical(iv, 1)
        y = plsc.bitcast(iv, jnp.float32)
        y = y * (1.5 - 0.5 * xv * y * y)
        y = y * (1.5 - 0.5 * xv * y * y)
        y = y * (1.5 - 0.5 * xv * y * y)
        dinv_v[sl] = y
        return c
    lax.fori_loop(0, NPT // L, newton, 0)

    pltpu.sync_copy(lin1_hbm.at[pl.ds(nbase, NPT)], g1s_v)

    def g1b(g, c):
        dvec = dinv_v[pl.ds(g * L, L)]
        for t in range(L):
            n = g * L + t
            g1s_v[n] = g1s_v[n] * dvec[t]
        return c
    lax.fori_loop(0, NPT // L, g1b, 0)
    pltpu.sync_copy(g1s_v, g1_sh.at[pl.ds(nbase, NPT)])
    plsc.subcore_barrier()

    def aggregate(g_sh, acc_sh):
        def grp(ci, cr):
            rb = ss * RPT + ci * KR
            pltpu.sync_copy(eiT_hbm.at[0, pl.ds(rb, KR)], sidx_v)
            pltpu.sync_copy(eiT_hbm.at[1, pl.ds(rb, KR)], didx_v)
            hs = []
            for j in range(KR):
                hs.append(pltpu.async_copy(
                    g_sh.at[sidx_v.at[j]], rows_v.at[j], gsem))
            sc = []
            for j in range(KR):
                hs[j].wait()
                sc.append(pltpu.async_copy(
                    rows_v.at[j], acc_sh.at[didx_v.at[j]], ssem, add=True))
            for j in range(KR):
                sc[j].wait()
            return cr
        lax.fori_loop(0, NGRP, grp, 0)

    aggregate(g1_sh, acc1_sh)
    plsc.subcore_barrier()

    pltpu.sync_copy(acc1_sh.at[pl.ds(nbase, NPT)], nbuf_v)
    b1vec = b1_v[...]
    w2rows = [w2_v[pl.ds(k * H, H)] for k in range(H)]

    def s4b(g, c):
        dvec = dinv_v[pl.ds(g * L, L)]
        for t in range(L):
            n = g * L + t
            s = dvec[t]
            acc = nbuf_v[n] + g1s_v[n]
            h1 = jnp.maximum(s * acc + b1vec, 0.0)
            lin2 = h1[0] * w2rows[0]
            for k in range(1, H):
                lin2 = lin2 + h1[k] * w2rows[k]
            g2s_v[n] = s * lin2
        return c
    lax.fori_loop(0, NPT // L, s4b, 0)
    pltpu.sync_copy(g2s_v, g2_sh.at[pl.ds(nbase, NPT)])
    lax.fori_loop(0, NPT, zn, 0)
    pltpu.sync_copy(nbuf_v, acc1_sh.at[pl.ds(nbase, NPT)])
    plsc.subcore_barrier()

    aggregate(g2_sh, acc1_sh)
    plsc.subcore_barrier()

    pltpu.sync_copy(acc1_sh.at[pl.ds(nbase, NPT)], nbuf_v)
    b2vec = b2_v[...]
    f1vec = f1_v[...]
    f2vec = f2_v[...]
    lanes = lax.iota(jnp.int32, L)

    def s6b(g, c):
        dvec = dinv_v[pl.ds(g * L, L)]
        a_acc = jnp.zeros((L,), jnp.float32)
        b_acc = jnp.zeros((L,), jnp.float32)
        for t in range(L):
            n = g * L + t
            h2 = dvec[t] * (nbuf_v[n] + g2s_v[n]) + b2vec
            a_acc = jnp.where(lanes == t, jnp.sum(h2 * f1vec), a_acc)
            b_acc = jnp.where(lanes == t, jnp.sum(h2 * f2vec), b_acc)
        sl = pl.ds(g * L, L)
        av_v[sl] = a_acc
        bv_v[sl] = b_acc
        return c
    lax.fori_loop(0, NPT // L, s6b, 0)
    pltpu.sync_copy(av_v, a_sh.at[pl.ds(nbase, NPT)])
    pltpu.sync_copy(bv_v, b_sh.at[pl.ds(nbase, NPT)])
    plsc.subcore_barrier()

    pltpu.sync_copy(a_sh, a_full)
    pltpu.sync_copy(b_sh, b_full)
    ebase = wid * EOUT_PT

    def ochunk(ci, cr):
        base = ebase + ci * CH
        pltpu.sync_copy(ei_hbm.at[0, pl.ds(base, CH)], idx_v)
        pltpu.sync_copy(ei_hbm.at[1, pl.ds(base, CH)], didx1_v)
        pltpu.sync_copy(c_hbm.at[pl.ds(base, CH)], c_v)

        def jb(j, c2):
            sl = pl.ds(j * L, L)
            av = plsc.load_gather(a_full, [idx_v[sl]])
            bv = plsc.load_gather(b_full, [didx1_v[sl]])
            p_v[sl] = av + bv + c_v[sl]
            return c2
        lax.fori_loop(0, CH // L, jb, 0)
        pltpu.sync_copy(p_v, pred_hbm.at[pl.ds(base, CH)])
        return cr
    lax.fori_loop(0, NOCH, ochunk, 0)


def kernel(x, edge_index, edge_attr, W1, b1, W2, b2, mW1, mb1, mW2, mb2,
           fcW, fcb):
    ei = edge_index.astype(jnp.int32)

    eye8 = jnp.eye(8, dtype=jnp.float32)
    c8, lin1_pad = _tc_dense(
        x, W1, edge_attr.reshape(E // 8, 128), jnp.kron(eye8, mW1),
        jnp.tile(mb1, 8).reshape(1, 128), mW2,
        fcW[2 * H:3 * H, :], mb2.reshape(1, H), fcb.reshape(1, 1),
    )
    c_edge = c8.reshape(E)

    eiT = ei.reshape(2, E // S, S)

    pred = _sc_mega(eiT, ei, lin1_pad,
                    W2.reshape(H * H), b1, b2,
                    fcW[0:H, 0], fcW[H:2 * H, 0], c_edge)
    return pred


# speedup vs baseline: 32.9898x; 1.0698x over previous
import functools

import jax
import jax.numpy as jnp
from jax import lax
from jax.experimental import pallas as pl
from jax.experimental.pallas import tpu as pltpu
from jax.experimental.pallas import tpu_sc as plsc

N = 10000
E = 320000
DF = 128
H = 16

NC, NS, L = 2, 16, 16
NW = NC * NS
NPAD = 10240
NPT = NPAD // NS
EPC = E
EPT = EPC // NS
CH = 2000
S = 125
KR = 8
RPT = EPT // S
NGRP = RPT // KR
EOUT_PT = E // NW
NOCH = EOUT_PT // CH


def _mesh():
    return plsc.VectorSubcoreMesh(core_axis_name="c", subcore_axis_name="s")


_SC_PARAMS = pltpu.CompilerParams(needs_layout_passes=False,
                                  use_tc_tiling_on_sc=False)


def _tc_dense(x, W1, ea8, W1big, b8, mW2, f3, mb2, fcb):
    R = E // 8
    RB = 8000

    def body(x_ref, w1_ref, ea_ref, w1b_ref, b8_ref, w2_ref, f3_ref,
             b2_ref, fcb_ref, c_ref, lin_ref):
        t8 = jnp.dot(ea_ref[...], w1b_ref[...],
                     preferred_element_type=jnp.float32) + b8_ref[...]
        t8 = jnp.maximum(t8, 0.0)
        w3 = jnp.dot(w2_ref[...], f3_ref[...],
                     preferred_element_type=jnp.float32)
        w3t = jnp.concatenate([w3] * 8, axis=0)
        r0 = lax.broadcasted_iota(jnp.int32, (128, 8), 0)
        r1 = lax.broadcasted_iota(jnp.int32, (128, 8), 1)
        wsel = jnp.where(r0 // H == r1, w3t, 0.0)
        cst = jnp.dot(b2_ref[...], f3_ref[...],
                      preferred_element_type=jnp.float32) + fcb_ref[...]
        c_ref[...] = jnp.dot(t8, wsel,
                             preferred_element_type=jnp.float32) + cst

        @pl.when(pl.program_id(0) == 0)
        def _():
            lin_ref[...] = jnp.zeros((NPAD, H), jnp.float32)
            lin_ref[pl.ds(0, N), :] = jnp.dot(
                x_ref[...], w1_ref[...], preferred_element_type=jnp.float32)

    return pl.pallas_call(
        body,
        grid=(R // RB,),
        in_specs=[
            pl.BlockSpec((N, DF), lambda i: (0, 0)),
            pl.BlockSpec((DF, H), lambda i: (0, 0)),
            pl.BlockSpec((RB, 128), lambda i: (i, 0)),
            pl.BlockSpec((128, 128), lambda i: (0, 0)),
            pl.BlockSpec((1, 128), lambda i: (0, 0)),
            pl.BlockSpec((H, H), lambda i: (0, 0)),
            pl.BlockSpec((H, 1), lambda i: (0, 0)),
            pl.BlockSpec((1, H), lambda i: (0, 0)),
            pl.BlockSpec((1, 1), lambda i: (0, 0)),
        ],
        out_specs=[
            pl.BlockSpec((RB, 8), lambda i: (i, 0)),
            pl.BlockSpec((NPAD, H), lambda i: (0, 0)),
        ],
        out_shape=[
            jax.ShapeDtypeStruct((R, 8), jnp.float32),
            jax.ShapeDtypeStruct((NPAD, H), jnp.float32),
        ],
    )(x, W1, ea8, W1big, b8, mW2, f3, mb2, fcb)


@functools.partial(
    pl.kernel,
    out_type=jax.ShapeDtypeStruct((E,), jnp.float32),
    mesh=_mesh(),
    compiler_params=_SC_PARAMS,
    scratch_types=[
        pltpu.VMEM((CH,), jnp.int32),
        pltpu.VMEM((CH,), jnp.int32),
        pltpu.VMEM((CH,), jnp.float32),
        pltpu.VMEM((CH,), jnp.float32),
        pltpu.VMEM((2, KR, S), jnp.int32),
        pltpu.VMEM((2, KR, S), jnp.int32),
        pltpu.VMEM((KR, S, H), jnp.float32),
        pltpu.VMEM((NPT, H), jnp.float32),
        pltpu.VMEM((NPT, H), jnp.float32),
        pltpu.VMEM((NPT, H), jnp.float32),
        pltpu.VMEM((NPT,), jnp.float32),
        pltpu.VMEM((NPT,), jnp.float32),
        pltpu.VMEM((NPT,), jnp.float32),
        pltpu.VMEM((NPT,), jnp.float32),
        pltpu.VMEM((H * H,), jnp.float32),
        pltpu.VMEM((H,), jnp.float32),
        pltpu.VMEM((H,), jnp.float32),
        pltpu.VMEM((H,), jnp.float32),
        pltpu.VMEM((H,), jnp.float32),
        pltpu.VMEM((NPAD,), jnp.float32),
        pltpu.VMEM((NPAD,), jnp.float32),
        pltpu.VMEM_SHARED((NS * NPAD,), jnp.float32),
        pltpu.VMEM_SHARED((NPAD, H), jnp.float32),
        pltpu.VMEM_SHARED((NPAD, H), jnp.float32),
        pltpu.VMEM_SHARED((NPAD, H), jnp.float32),
        pltpu.VMEM_SHARED((NPAD,), jnp.float32),
        pltpu.VMEM_SHARED((NPAD,), jnp.float32),
        pltpu.SemaphoreType.DMA,
        pltpu.SemaphoreType.DMA,
        pltpu.SemaphoreType.DMA,
    ],
)
def _sc_mega(eiT_hbm, ei_hbm, lin1_hbm,
             w2_hbm, b1_hbm, b2_hbm, f1_hbm, f2_hbm, c_hbm, pred_hbm,
             idx_v, didx1_v, c_v, p_v, sidx_v, didx_v, rows_v,
             nbuf_v, g1s_v, g2s_v, dinv_v, tmp_v, av_v, bv_v,
             w2_v, b1_v, b2_v, f1_v, f2_v, a_full, b_full,
             part_sh, g1_sh, acc1_sh, g2_sh, a_sh, b_sh,
             gsem, ssem, isem):
    cc = lax.axis_index("c")
    ss = lax.axis_index("s")
    wid = ss * NC + cc
    nbase = ss * NPT
    zeros = jnp.zeros((L,), jnp.float32)
    ones = jnp.ones((L,), jnp.float32)

    pltpu.sync_copy(w2_hbm, w2_v)
    pltpu.sync_copy(b1_hbm, b1_v)
    pltpu.sync_copy(b2_hbm, b2_v)
    pltpu.sync_copy(f1_hbm, f1_v)
    pltpu.sync_copy(f2_hbm, f2_v)

    zrow = jnp.zeros((L,), jnp.float32)

    def zn(n, c):
        nbuf_v[n] = zrow
        return c
    lax.fori_loop(0, NPT, zn, 0)
    pltpu.sync_copy(nbuf_v, acc1_sh.at[pl.ds(nbase, NPT)])

    def zb(i, c):
        a_full[pl.ds(i * L, L)] = zeros
        return c
    lax.fori_loop(0, NPAD // L, zb, 0)

    def dchunk(ci, c):
        pltpu.sync_copy(ei_hbm.at[1, pl.ds(ss * EPT + ci * CH, CH)], idx_v)

        def scat(j, c2):
            plsc.addupdate_scatter(a_full, [idx_v[pl.ds(j * L, L)]], ones)
            return c2
        lax.fori_loop(0, CH // L, scat, 0)
        return c
    lax.fori_loop(0, EPT // CH, dchunk, 0)
    pltpu.sync_copy(a_full, part_sh.at[pl.ds(ss * NPAD, NPAD)])
    plsc.subcore_barrier()

    def z2(i, c):
        dinv_v[pl.ds(i * L, L)] = zeros
        return c
    lax.fori_loop(0, NPT // L, z2, 0)
    for t in range(NS):
        pltpu.sync_copy(part_sh.at[pl.ds(t * NPAD + nbase, NPT)], tmp_v)

        def ab(i, c):
            sl = pl.ds(i * L, L)
            dinv_v[sl] = dinv_v[sl] + tmp_v[sl]
            return c
        lax.fori_loop(0, NPT // L, ab, 0)

    def newton(i, c):
        sl = pl.ds(i * L, L)
        xv = dinv_v[sl] + 1.0
        iv = plsc.bitcast(xv, jnp.int32)
        iv = 0x5F3759DF - lax.shift_right_logical(iv, 1)
        y = plsc.bitcast(iv, jnp.float32)
        y = y * (1.5 - 0.5 * xv * y * y)
        y = y * (1.5 - 0.5 * xv * y * y)
        y = y * (1.5 - 0.5 * xv * y * y)
        dinv_v[sl] = y
        return c
    lax.fori_loop(0, NPT // L, newton, 0)

    pltpu.sync_copy(lin1_hbm.at[pl.ds(nbase, NPT)], g1s_v)

    def g1b(g, c):
        dvec = dinv_v[pl.ds(g * L, L)]
        for t in range(L):
            n = g * L + t
            g1s_v[n] = g1s_v[n] * dvec[t]
        return c
    lax.fori_loop(0, NPT // L, g1b, 0)
    pltpu.sync_copy(g1s_v, g1_sh.at[pl.ds(nbase, NPT)])
    plsc.subcore_barrier()

    def aggregate(g_sh, acc_sh):
        rb0 = ss * RPT
        pltpu.async_copy(eiT_hbm.at[0, pl.ds(rb0, KR)], sidx_v.at[0], isem)
        pltpu.async_copy(eiT_hbm.at[1, pl.ds(rb0, KR)], didx_v.at[0], isem)

        def grp(ci, cr):
            cur = lax.rem(ci, 2)
            nxt = lax.rem(ci + 1, 2)
            pltpu.make_async_copy(eiT_hbm.at[0, pl.ds(rb0, KR)],
                                  sidx_v.at[0], isem).wait()
            pltpu.make_async_copy(eiT_hbm.at[1, pl.ds(rb0, KR)],
                                  didx_v.at[0], isem).wait()
            rbn = rb0 + jnp.minimum(ci + 1, NGRP - 1) * KR
            pltpu.async_copy(eiT_hbm.at[0, pl.ds(rbn, KR)],
                             sidx_v.at[nxt], isem)
            pltpu.async_copy(eiT_hbm.at[1, pl.ds(rbn, KR)],
                             didx_v.at[nxt], isem)
            hs = []
            for j in range(KR):
                hs.append(pltpu.async_copy(
                    g_sh.at[sidx_v.at[cur].at[j]], rows_v.at[j], gsem))
            sc = []
            for j in range(KR):
                hs[j].wait()
                sc.append(pltpu.async_copy(
                    rows_v.at[j], acc_sh.at[didx_v.at[cur].at[j]],
                    ssem, add=True))
            for j in range(KR):
                sc[j].wait()
            return cr
        lax.fori_loop(0, NGRP, grp, 0)
        pltpu.make_async_copy(eiT_hbm.at[0, pl.ds(rb0, KR)],
                              sidx_v.at[0], isem).wait()
        pltpu.make_async_copy(eiT_hbm.at[1, pl.ds(rb0, KR)],
                              didx_v.at[0], isem).wait()

    aggregate(g1_sh, acc1_sh)
    plsc.subcore_barrier()

    pltpu.sync_copy(acc1_sh.at[pl.ds(nbase, NPT)], nbuf_v)
    b1vec = b1_v[...]
    w2rows = [w2_v[pl.ds(k * H, H)] for k in range(H)]

    def s4b(g, c):
        dvec = dinv_v[pl.ds(g * L, L)]
        for t in range(L):
            n = g * L + t
            s = dvec[t]
            acc = nbuf_v[n] + g1s_v[n]
            h1 = jnp.maximum(s * acc + b1vec, 0.0)
            lin2 = h1[0] * w2rows[0]
            for k in range(1, H):
                lin2 = lin2 + h1[k] * w2rows[k]
            g2s_v[n] = s * lin2
        return c
    lax.fori_loop(0, NPT // L, s4b, 0)
    pltpu.sync_copy(g2s_v, g2_sh.at[pl.ds(nbase, NPT)])
    lax.fori_loop(0, NPT, zn, 0)
    pltpu.sync_copy(nbuf_v, acc1_sh.at[pl.ds(nbase, NPT)])
    plsc.subcore_barrier()

    aggregate(g2_sh, acc1_sh)
    plsc.subcore_barrier()

    pltpu.sync_copy(acc1_sh.at[pl.ds(nbase, NPT)], nbuf_v)
    b2vec = b2_v[...]
    f1vec = f1_v[...]
    f2vec = f2_v[...]
    lanes = lax.iota(jnp.int32, L)

    def s6b(g, c):
        dvec = dinv_v[pl.ds(g * L, L)]
        a_acc = jnp.zeros((L,), jnp.float32)
        b_acc = jnp.zeros((L,), jnp.float32)
        for t in range(L):
            n = g * L + t
            h2 = dvec[t] * (nbuf_v[n] + g2s_v[n]) + b2vec
            a_acc = jnp.where(lanes == t, jnp.sum(h2 * f1vec), a_acc)
            b_acc = jnp.where(lanes == t, jnp.sum(h2 * f2vec), b_acc)
        sl = pl.ds(g * L, L)
        av_v[sl] = a_acc
        bv_v[sl] = b_acc
        return c
    lax.fori_loop(0, NPT // L, s6b, 0)
    pltpu.sync_copy(av_v, a_sh.at[pl.ds(nbase, NPT)])
    pltpu.sync_copy(bv_v, b_sh.at[pl.ds(nbase, NPT)])
    plsc.subcore_barrier()

    pltpu.sync_copy(a_sh, a_full)
    pltpu.sync_copy(b_sh, b_full)
    ebase = wid * EOUT_PT

    def ochunk(ci, cr):
        base = ebase + ci * CH
        pltpu.sync_copy(ei_hbm.at[0, pl.ds(base, CH)], idx_v)
        pltpu.sync_copy(ei_hbm.at[1, pl.ds(base, CH)], didx1_v)
        pltpu.sync_copy(c_hbm.at[pl.ds(base, CH)], c_v)

        def jb(j, c2):
            sl = pl.ds(j * L, L)
            av = plsc.load_gather(a_full, [idx_v[sl]])
            bv = plsc.load_gather(b_full, [didx1_v[sl]])
            p_v[sl] = av + bv + c_v[sl]
            return c2
        lax.fori_loop(0, CH // L, jb, 0)
        pltpu.sync_copy(p_v, pred_hbm.at[pl.ds(base, CH)])
        return cr
    lax.fori_loop(0, NOCH, ochunk, 0)


def kernel(x, edge_index, edge_attr, W1, b1, W2, b2, mW1, mb1, mW2, mb2,
           fcW, fcb):
    ei = edge_index.astype(jnp.int32)

    eye8 = jnp.eye(8, dtype=jnp.float32)
    c8, lin1_pad = _tc_dense(
        x, W1, edge_attr.reshape(E // 8, 128), jnp.kron(eye8, mW1),
        jnp.tile(mb1, 8).reshape(1, 128), mW2,
        fcW[2 * H:3 * H, :], mb2.reshape(1, H), fcb.reshape(1, 1),
    )
    c_edge = c8.reshape(E)

    eiT = ei.reshape(2, E // S, S)

    pred = _sc_mega(eiT, ei, lin1_pad,
                    W2.reshape(H * H), b1, b2,
                    fcW[0:H, 0], fcW[H:2 * H, 0], c_edge)
    return pred


# speedup vs baseline: 35.0197x; 1.0615x over previous
import functools

import jax
import jax.numpy as jnp
from jax import lax
from jax.experimental import pallas as pl
from jax.experimental.pallas import tpu as pltpu
from jax.experimental.pallas import tpu_sc as plsc

N = 10000
E = 320000
DF = 128
H = 16

NC, NS, L = 2, 16, 16
NW = NC * NS
NPAD = 10240
NPT = NPAD // NS
EPC = E
EPT = EPC // NS
CH = 2000
S = 125
KR = 8
RPT = EPT // S
NGRP = RPT // KR
EOUT_PT = E // NW
NOCH = EOUT_PT // CH


def _mesh():
    return plsc.VectorSubcoreMesh(core_axis_name="c", subcore_axis_name="s")


_SC_PARAMS = pltpu.CompilerParams(needs_layout_passes=False,
                                  use_tc_tiling_on_sc=False)


def _tc_dense(x, W1, ea8, W1big, b8, mW2, f3, mb2, fcb):
    R = E // 8
    RB = 8000

    def body(x_ref, w1_ref, ea_ref, w1b_ref, b8_ref, w2_ref, f3_ref,
             b2_ref, fcb_ref, c_ref, lin_ref):
        t8 = jnp.dot(ea_ref[...], w1b_ref[...],
                     preferred_element_type=jnp.float32) + b8_ref[...]
        t8 = jnp.maximum(t8, 0.0)
        w3 = jnp.dot(w2_ref[...], f3_ref[...],
                     preferred_element_type=jnp.float32)
        w3t = jnp.concatenate([w3] * 8, axis=0)
        r0 = lax.broadcasted_iota(jnp.int32, (128, 8), 0)
        r1 = lax.broadcasted_iota(jnp.int32, (128, 8), 1)
        wsel = jnp.where(r0 // H == r1, w3t, 0.0)
        cst = jnp.dot(b2_ref[...], f3_ref[...],
                      preferred_element_type=jnp.float32) + fcb_ref[...]
        c_ref[...] = jnp.dot(t8, wsel,
                             preferred_element_type=jnp.float32) + cst

        @pl.when(pl.program_id(0) == 0)
        def _():
            lin_ref[...] = jnp.zeros((NPAD, H), jnp.float32)
            lin_ref[pl.ds(0, N), :] = jnp.dot(
                x_ref[...], w1_ref[...], preferred_element_type=jnp.float32)

    return pl.pallas_call(
        body,
        grid=(R // RB,),
        in_specs=[
            pl.BlockSpec((N, DF), lambda i: (0, 0)),
            pl.BlockSpec((DF, H), lambda i: (0, 0)),
            pl.BlockSpec((RB, 128), lambda i: (i, 0)),
            pl.BlockSpec((128, 128), lambda i: (0, 0)),
            pl.BlockSpec((1, 128), lambda i: (0, 0)),
            pl.BlockSpec((H, H), lambda i: (0, 0)),
            pl.BlockSpec((H, 1), lambda i: (0, 0)),
            pl.BlockSpec((1, H), lambda i: (0, 0)),
            pl.BlockSpec((1, 1), lambda i: (0, 0)),
        ],
        out_specs=[
            pl.BlockSpec((RB, 8), lambda i: (i, 0)),
            pl.BlockSpec((NPAD, H), lambda i: (0, 0)),
        ],
        out_shape=[
            jax.ShapeDtypeStruct((R, 8), jnp.float32),
            jax.ShapeDtypeStruct((NPAD, H), jnp.float32),
        ],
    )(x, W1, ea8, W1big, b8, mW2, f3, mb2, fcb)


@functools.partial(
    pl.kernel,
    out_type=jax.ShapeDtypeStruct((E,), jnp.float32),
    mesh=_mesh(),
    compiler_params=_SC_PARAMS,
    scratch_types=[
        pltpu.VMEM((CH,), jnp.int32),
        pltpu.VMEM((CH,), jnp.int32),
        pltpu.VMEM((CH,), jnp.float32),
        pltpu.VMEM((CH,), jnp.float32),
        pltpu.VMEM((2, KR, S), jnp.int32),
        pltpu.VMEM((2, KR, S), jnp.int32),
        pltpu.VMEM((2, KR // 2, S, H), jnp.float32),
        pltpu.VMEM((NPT, H), jnp.float32),
        pltpu.VMEM((NPT, H), jnp.float32),
        pltpu.VMEM((NPT, H), jnp.float32),
        pltpu.VMEM((NPT,), jnp.float32),
        pltpu.VMEM((NPT,), jnp.float32),
        pltpu.VMEM((NPT,), jnp.float32),
        pltpu.VMEM((NPT,), jnp.float32),
        pltpu.VMEM((H * H,), jnp.float32),
        pltpu.VMEM((H,), jnp.float32),
        pltpu.VMEM((H,), jnp.float32),
        pltpu.VMEM((H,), jnp.float32),
        pltpu.VMEM((H,), jnp.float32),
        pltpu.VMEM((NPAD,), jnp.float32),
        pltpu.VMEM((NPAD,), jnp.float32),
        pltpu.VMEM_SHARED((NS * NPAD,), jnp.float32),
        pltpu.VMEM_SHARED((NPAD, H), jnp.float32),
        pltpu.VMEM_SHARED((NPAD, H), jnp.float32),
        pltpu.VMEM_SHARED((NPAD, H), jnp.float32),
        pltpu.VMEM_SHARED((NPAD,), jnp.float32),
        pltpu.VMEM_SHARED((NPAD,), jnp.float32),
        pltpu.SemaphoreType.DMA,
        pltpu.SemaphoreType.DMA,
        pltpu.SemaphoreType.DMA,
        pltpu.SemaphoreType.DMA,
    ],
)
def _sc_mega(eiT_hbm, ei_hbm, lin1_hbm,
             w2_hbm, b1_hbm, b2_hbm, f1_hbm, f2_hbm, c_hbm, pred_hbm,
             idx_v, didx1_v, c_v, p_v, sidx_v, didx_v, rows_v,
             nbuf_v, g1s_v, g2s_v, dinv_v, tmp_v, av_v, bv_v,
             w2_v, b1_v, b2_v, f1_v, f2_v, a_full, b_full,
             part_sh, g1_sh, acc1_sh, g2_sh, a_sh, b_sh,
             gsem, ssem0, ssem1, isem):
    cc = lax.axis_index("c")
    ss = lax.axis_index("s")
    wid = ss * NC + cc
    nbase = ss * NPT
    zeros = jnp.zeros((L,), jnp.float32)
    ones = jnp.ones((L,), jnp.float32)

    pltpu.sync_copy(w2_hbm, w2_v)
    pltpu.sync_copy(b1_hbm, b1_v)
    pltpu.sync_copy(b2_hbm, b2_v)
    pltpu.sync_copy(f1_hbm, f1_v)
    pltpu.sync_copy(f2_hbm, f2_v)

    zrow = jnp.zeros((L,), jnp.float32)

    def zn(n, c):
        nbuf_v[n] = zrow
        return c
    lax.fori_loop(0, NPT, zn, 0)
    pltpu.sync_copy(nbuf_v, acc1_sh.at[pl.ds(nbase, NPT)])

    def zb(i, c):
        a_full[pl.ds(i * L, L)] = zeros
        return c
    lax.fori_loop(0, NPAD // L, zb, 0)

    def dchunk(ci, c):
        pltpu.sync_copy(ei_hbm.at[1, pl.ds(ss * EPT + ci * CH, CH)], idx_v)

        def scat(j, c2):
            plsc.addupdate_scatter(a_full, [idx_v[pl.ds(j * L, L)]], ones)
            return c2
        lax.fori_loop(0, CH // L, scat, 0)
        return c
    lax.fori_loop(0, EPT // CH, dchunk, 0)
    pltpu.sync_copy(a_full, part_sh.at[pl.ds(ss * NPAD, NPAD)])
    plsc.subcore_barrier()

    def z2(i, c):
        dinv_v[pl.ds(i * L, L)] = zeros
        return c
    lax.fori_loop(0, NPT // L, z2, 0)
    for t in range(NS):
        pltpu.sync_copy(part_sh.at[pl.ds(t * NPAD + nbase, NPT)], tmp_v)

        def ab(i, c):
            sl = pl.ds(i * L, L)
            dinv_v[sl] = dinv_v[sl] + tmp_v[sl]
            return c
        lax.fori_loop(0, NPT // L, ab, 0)

    def newton(i, c):
        sl = pl.ds(i * L, L)
        xv = dinv_v[sl] + 1.0
        iv = plsc.bitcast(xv, jnp.int32)
        iv = 0x5F3759DF - lax.shift_right_logical(iv, 1)
        y = plsc.bitcast(iv, jnp.float32)
        y = y * (1.5 - 0.5 * xv * y * y)
        y = y * (1.5 - 0.5 * xv * y * y)
        y = y * (1.5 - 0.5 * xv * y * y)
        dinv_v[sl] = y
        return c
    lax.fori_loop(0, NPT // L, newton, 0)

    pltpu.sync_copy(lin1_hbm.at[pl.ds(nbase, NPT)], g1s_v)

    def g1b(g, c):
        dvec = dinv_v[pl.ds(g * L, L)]
        for t in range(L):
            n = g * L + t
            g1s_v[n] = g1s_v[n] * dvec[t]
        return c
    lax.fori_loop(0, NPT // L, g1b, 0)
    pltpu.sync_copy(g1s_v, g1_sh.at[pl.ds(nbase, NPT)])
    plsc.subcore_barrier()

    HK = KR // 2

    def aggregate(g_sh, acc_sh):
        rb0 = ss * RPT
        ssems = (ssem0, ssem1)

        def stage_idx(slot, ci):
            rb = rb0 + ci * KR
            pltpu.async_copy(eiT_hbm.at[0, pl.ds(rb, KR)],
                             sidx_v.at[slot], isem)
            pltpu.async_copy(eiT_hbm.at[1, pl.ds(rb, KR)],
                             didx_v.at[slot], isem)

        def wait_idx(slot):
            pltpu.make_async_copy(eiT_hbm.at[0, pl.ds(rb0, KR)],
                                  sidx_v.at[slot], isem).wait()
            pltpu.make_async_copy(eiT_hbm.at[1, pl.ds(rb0, KR)],
                                  didx_v.at[slot], isem).wait()

        def half_batch(cur, h, drain):
            if drain:
                for j in range(HK):
                    pltpu.make_async_copy(
                        rows_v.at[h].at[j],
                        acc_sh.at[didx_v.at[cur].at[h * HK + j]],
                        ssems[h]).wait()
            hs = []
            for j in range(HK):
                hs.append(pltpu.async_copy(
                    g_sh.at[sidx_v.at[cur].at[h * HK + j]],
                    rows_v.at[h].at[j], gsem))
            for j in range(HK):
                hs[j].wait()
                pltpu.async_copy(rows_v.at[h].at[j],
                                 acc_sh.at[didx_v.at[cur].at[h * HK + j]],
                                 ssems[h], add=True)

        stage_idx(0, 0)
        wait_idx(0)
        stage_idx(1, 1)
        half_batch(0, 0, False)
        half_batch(0, 1, False)

        def grp(ci, cr):
            cur = lax.rem(ci, 2)
            nxt = lax.rem(ci + 1, 2)
            wait_idx_dyn(cur)
            rbn = rb0 + jnp.minimum(ci + 1, NGRP - 1) * KR
            pltpu.async_copy(eiT_hbm.at[0, pl.ds(rbn, KR)],
                             sidx_v.at[nxt], isem)
            pltpu.async_copy(eiT_hbm.at[1, pl.ds(rbn, KR)],
                             didx_v.at[nxt], isem)
            half_batch_dyn(cur, 0)
            half_batch_dyn(cur, 1)
            return cr

        def wait_idx_dyn(slot):
            pltpu.make_async_copy(eiT_hbm.at[0, pl.ds(rb0, KR)],
                                  sidx_v.at[slot], isem).wait()
            pltpu.make_async_copy(eiT_hbm.at[1, pl.ds(rb0, KR)],
                                  didx_v.at[slot], isem).wait()

        def half_batch_dyn(cur, h):
            for j in range(HK):
                pltpu.make_async_copy(
                    rows_v.at[h].at[j],
                    acc_sh.at[didx_v.at[cur].at[h * HK + j]],
                    ssems[h]).wait()
            hs = []
            for j in range(HK):
                hs.append(pltpu.async_copy(
                    g_sh.at[sidx_v.at[cur].at[h * HK + j]],
                    rows_v.at[h].at[j], gsem))
            for j in range(HK):
                hs[j].wait()
                pltpu.async_copy(rows_v.at[h].at[j],
                                 acc_sh.at[didx_v.at[cur].at[h * HK + j]],
                                 ssems[h], add=True)

        lax.fori_loop(1, NGRP, grp, 0)
        pltpu.make_async_copy(eiT_hbm.at[0, pl.ds(rb0, KR)],
                              sidx_v.at[0], isem).wait()
        pltpu.make_async_copy(eiT_hbm.at[1, pl.ds(rb0, KR)],
                              didx_v.at[0], isem).wait()
        for h in range(2):
            for j in range(HK):
                pltpu.make_async_copy(
                    rows_v.at[h].at[j],
                    acc_sh.at[didx_v.at[0].at[h * HK + j]],
                    ssems[h]).wait()

    aggregate(g1_sh, acc1_sh)
    plsc.subcore_barrier()

    pltpu.sync_copy(acc1_sh.at[pl.ds(nbase, NPT)], nbuf_v)
    b1vec = b1_v[...]
    w2rows = [w2_v[pl.ds(k * H, H)] for k in range(H)]

    def s4b(g, c):
        dvec = dinv_v[pl.ds(g * L, L)]
        for t in range(L):
            n = g * L + t
            s = dvec[t]
            acc = nbuf_v[n] + g1s_v[n]
            h1 = jnp.maximum(s * acc + b1vec, 0.0)
            lin2 = h1[0] * w2rows[0]
            for k in range(1, H):
                lin2 = lin2 + h1[k] * w2rows[k]
            g2s_v[n] = s * lin2
        return c
    lax.fori_loop(0, NPT // L, s4b, 0)
    pltpu.sync_copy(g2s_v, g2_sh.at[pl.ds(nbase, NPT)])
    lax.fori_loop(0, NPT, zn, 0)
    pltpu.sync_copy(nbuf_v, acc1_sh.at[pl.ds(nbase, NPT)])
    plsc.subcore_barrier()

    aggregate(g2_sh, acc1_sh)
    plsc.subcore_barrier()

    pltpu.sync_copy(acc1_sh.at[pl.ds(nbase, NPT)], nbuf_v)
    b2vec = b2_v[...]
    f1vec = f1_v[...]
    f2vec = f2_v[...]
    lanes = lax.iota(jnp.int32, L)

    def s6b(g, c):
        dvec = dinv_v[pl.ds(g * L, L)]
        a_acc = jnp.zeros((L,), jnp.float32)
        b_acc = jnp.zeros((L,), jnp.float32)
        for t in range(L):
            n = g * L + t
            h2 = dvec[t] * (nbuf_v[n] + g2s_v[n]) + b2vec
            a_acc = jnp.where(lanes == t, jnp.sum(h2 * f1vec), a_acc)
            b_acc = jnp.where(lanes == t, jnp.sum(h2 * f2vec), b_acc)
        sl = pl.ds(g * L, L)
        av_v[sl] = a_acc
        bv_v[sl] = b_acc
        return c
    lax.fori_loop(0, NPT // L, s6b, 0)
    pltpu.sync_copy(av_v, a_sh.at[pl.ds(nbase, NPT)])
    pltpu.sync_copy(bv_v, b_sh.at[pl.ds(nbase, NPT)])
    plsc.subcore_barrier()

    pltpu.sync_copy(a_sh, a_full)
    pltpu.sync_copy(b_sh, b_full)
    ebase = wid * EOUT_PT

    def ochunk(ci, cr):
        base = ebase + ci * CH
        pltpu.sync_copy(ei_hbm.at[0, pl.ds(base, CH)], idx_v)
        pltpu.sync_copy(ei_hbm.at[1, pl.ds(base, CH)], didx1_v)
        pltpu.sync_copy(c_hbm.at[pl.ds(base, CH)], c_v)

        def jb(j, c2):
            sl = pl.ds(j * L, L)
            av = plsc.load_gather(a_full, [idx_v[sl]])
            bv = plsc.load_gather(b_full, [didx1_v[sl]])
            p_v[sl] = av + bv + c_v[sl]
            return c2
        lax.fori_loop(0, CH // L, jb, 0)
        pltpu.sync_copy(p_v, pred_hbm.at[pl.ds(base, CH)])
        return cr
    lax.fori_loop(0, NOCH, ochunk, 0)


def kernel(x, edge_index, edge_attr, W1, b1, W2, b2, mW1, mb1, mW2, mb2,
           fcW, fcb):
    ei = edge_index.astype(jnp.int32)

    eye8 = jnp.eye(8, dtype=jnp.float32)
    c8, lin1_pad = _tc_dense(
        x, W1, edge_attr.reshape(E // 8, 128), jnp.kron(eye8, mW1),
        jnp.tile(mb1, 8).reshape(1, 128), mW2,
        fcW[2 * H:3 * H, :], mb2.reshape(1, H), fcb.reshape(1, 1),
    )
    c_edge = c8.reshape(E)

    eiT = ei.reshape(2, E // S, S)

    pred = _sc_mega(eiT, ei, lin1_pad,
                    W2.reshape(H * H), b1, b2,
                    fcW[0:H, 0], fcW[H:2 * H, 0], c_edge)
    return pred
